# baseline (speedup 1.0000x reference)
import functools
import jax
import jax.numpy as jnp
import numpy as np
from jax.experimental import pallas as pl
from jax.experimental.pallas import tpu as pltpu

LD = 128
EXT = 144


def _ceil(a, b):
    return (a + b - 1) // b


def _pre_body(h_ref, pos_ref, wa_ref, wb_ref, wp_ref, b1_ref, u_ref, v_ref):
    h = h_ref[...]
    posp = pos_ref[...] @ wp_ref[...]
    u_ref[...] = h @ wa_ref[...] + posp + b1_ref[...]
    v_ref[...] = h @ wb_ref[...] - posp


def _tc_pre(h, pos, wa, wb, wp, b1, blk=512):
    n = h.shape[0]
    grid = (_ceil(n, blk),)
    return pl.pallas_call(
        _pre_body,
        grid=grid,
        in_specs=[
            pl.BlockSpec((blk, LD), lambda i: (i, 0)),
            pl.BlockSpec((blk, 8), lambda i: (i, 0)),
            pl.BlockSpec((LD, LD), lambda i: (0, 0)),
            pl.BlockSpec((LD, LD), lambda i: (0, 0)),
            pl.BlockSpec((8, LD), lambda i: (0, 0)),
            pl.BlockSpec((1, LD), lambda i: (0, 0)),
        ],
        out_specs=[
            pl.BlockSpec((blk, LD), lambda i: (i, 0)),
            pl.BlockSpec((blk, LD), lambda i: (i, 0)),
        ],
        out_shape=[
            jax.ShapeDtypeStruct((n, LD), jnp.float32),
            jax.ShapeDtypeStruct((n, LD), jnp.float32),
        ],
    )(h, pos, wa, wb, wp, b1)


def _node_body(h_ref, s_ref, w2e_ref, wna_ref, wnb_ref, bn1_ref, wn2_ref,
               bn2_ref, res_ref, o_ref):
    h = h_ref[...]
    aggr = s_ref[...] @ w2e_ref[...]
    z = jnp.maximum(h @ wna_ref[...] + aggr @ wnb_ref[...] + bn1_ref[...], 0.0)
    o_ref[...] = h + z @ wn2_ref[...] + bn2_ref[...] + res_ref[...]


def _tc_node(h, s_ext, w2e_ext, wna, wnb, bn1, wn2, bn2, res, blk=512):
    n = h.shape[0]
    grid = (_ceil(n, blk),)
    return pl.pallas_call(
        _node_body,
        grid=grid,
        in_specs=[
            pl.BlockSpec((blk, LD), lambda i: (i, 0)),
            pl.BlockSpec((blk, EXT), lambda i: (i, 0)),
            pl.BlockSpec((EXT, LD), lambda i: (0, 0)),
            pl.BlockSpec((LD, LD), lambda i: (0, 0)),
            pl.BlockSpec((LD, LD), lambda i: (0, 0)),
            pl.BlockSpec((1, LD), lambda i: (0, 0)),
            pl.BlockSpec((LD, LD), lambda i: (0, 0)),
            pl.BlockSpec((1, LD), lambda i: (0, 0)),
            pl.BlockSpec((blk, LD), lambda i: (i, 0)),
        ],
        out_specs=pl.BlockSpec((blk, LD), lambda i: (i, 0)),
        out_shape=jax.ShapeDtypeStruct((n, LD), jnp.float32),
    )(h, s_ext, w2e_ext, wna, wnb, bn1, wn2, bn2, res)


def _edge_pass(u, v, pos3, src, dst, wd):
    n = u.shape[0]
    diff = pos3[src] - pos3[dst]
    dist = jnp.sqrt(jnp.sum(diff * diff, axis=-1, keepdims=True) + 1e-12)
    pre = u[src] + v[dst] + dist * wd[None, :]
    m = jnp.maximum(pre, 0.0)
    ones = jnp.ones((m.shape[0], 1), jnp.float32)
    pad = jnp.zeros((m.shape[0], EXT - LD - 1), jnp.float32)
    rows = jnp.concatenate([m, ones, pad], axis=-1)
    return jax.ops.segment_sum(rows, dst, num_segments=n)


def _prep_level(w, src, dst):
    n = w.shape[0]
    deg = jax.ops.segment_sum(jnp.ones(src.shape[0], jnp.float32), src,
                              num_segments=n)
    deg = jnp.maximum(deg, 1.0)
    w_to_send = (w / deg)[src]
    aggr_w = jax.ops.segment_sum(w_to_send, dst, num_segments=n) + 1e-12
    ec = w_to_send / aggr_w[dst]
    return ec, aggr_w


def _pool(x, src, dst, ec, n):
    return jax.ops.segment_sum(ec[:, None] * x[src], dst, num_segments=n)


def _unpool_conv(h_small, m_ids, src, dst, ec, n):
    u = jnp.zeros((n, h_small.shape[-1]), h_small.dtype).at[m_ids].set(h_small)
    return jax.ops.segment_sum(ec[:, None] * u[dst], src, num_segments=n)


def _split_gmp_params(p):
    w1e = p["edge"]["W1"]
    wa = w1e[:LD]
    wb = w1e[LD:2 * LD]
    wp = jnp.pad(w1e[2 * LD:2 * LD + 3], ((0, 5), (0, 0)))
    wd = w1e[2 * LD + 3]
    b1 = p["edge"]["b1"][None, :]
    w2e_ext = jnp.concatenate(
        [p["edge"]["W2"], p["edge"]["b2"][None, :],
         jnp.zeros((EXT - LD - 1, LD), jnp.float32)], axis=0)
    wn1 = p["node"]["W1"]
    wna = wn1[:LD]
    wnb = wn1[LD:]
    bn1 = p["node"]["b1"][None, :]
    wn2 = p["node"]["W2"]
    bn2 = p["node"]["b2"][None, :]
    return wa, wb, wp, wd, b1, w2e_ext, wna, wnb, bn1, wn2, bn2


def _gmp(p, h, pos3, pos8, src, dst, res=None):
    wa, wb, wp, wd, b1, w2e_ext, wna, wnb, bn1, wn2, bn2 = _split_gmp_params(p)
    u, v = _tc_pre(h, pos8, wa, wb, wp, b1)
    s_ext = _edge_pass(u, v, pos3, src, dst, wd)
    if res is None:
        res = jnp.zeros_like(h)
    return _tc_node(h, s_ext, w2e_ext, wna, wnb, bn1, wn2, bn2, res)


def kernel(h, pos, params, m_gs_0, m_gs_1, m_gs_2, m_ids_0, m_ids_1):
    m_gs = [m_gs_0, m_gs_1, m_gs_2]
    m_ids = [m_ids_0, m_ids_1]
    pos3 = pos
    w = jnp.ones((h.shape[0],), jnp.float32)

    down_outs, down_pos3, ecs = [], [], []
    for i in range(2):
        src, dst = m_gs[i][0], m_gs[i][1]
        n = h.shape[0]
        pos8 = jnp.pad(pos3, ((0, 0), (0, 8 - pos3.shape[1])))
        h = _gmp(params["down"][i], h, pos3, pos8, src, dst)
        down_outs.append(h)
        down_pos3.append(pos3)
        ec, aggr_w = _prep_level(w, src, dst)
        ecs.append(ec)
        h = _pool(h, src, dst, ec, n)[m_ids[i]]
        pos3 = _pool(pos3, src, dst, ec, n)[m_ids[i]]
        w = aggr_w[m_ids[i]]

    src, dst = m_gs[2][0], m_gs[2][1]
    pos8 = jnp.pad(pos3, ((0, 0), (0, 8 - pos3.shape[1])))
    h = _gmp(params["bottom"], h, pos3, pos8, src, dst)

    for i in range(2):
        li = 1 - i
        src, dst = m_gs[li][0], m_gs[li][1]
        n = down_outs[li].shape[0]
        p3 = down_pos3[li]
        h = _unpool_conv(h, m_ids[li], src, dst, ecs[li], n)
        pos8 = jnp.pad(p3, ((0, 0), (0, 8 - p3.shape[1])))
        h = _gmp(params["up"][i], h, p3, pos8, src, dst, res=down_outs[li])
    return h


# speedup vs baseline: 2.0003x; 2.0003x over previous
import functools
import jax
import jax.numpy as jnp
from jax import lax
from jax.experimental import pallas as pl
from jax.experimental.pallas import tpu as pltpu
from jax.experimental.pallas import tpu_sc as plsc

LD = 128
NCORE = 2
NSUB = 16
NW = NCORE * NSUB
ECHUNK = 128
F32 = jnp.float32
I32 = jnp.int32

NS_TAB = {10000: 10240, 5000: 5120, 2500: 2560}


def _ceil(a, b):
    return (a + b - 1) // b


def _sc_mesh():
    return plsc.VectorSubcoreMesh(core_axis_name="c", subcore_axis_name="s",
                                  num_cores=NCORE, num_subcores=NSUB)


def _wid():
    return lax.axis_index("s") * NCORE + lax.axis_index("c")


def _fill_rows(buf_ref, val):
    def zrow(i, _):
        for k in range(buf_ref.shape[1] // 16):
            buf_ref[i, pl.ds(16 * k, 16)] = jnp.full((16,), val, F32)
        return ()
    lax.fori_loop(0, buf_ref.shape[0], zrow, ())


def _fill_flat(buf_ref, val):
    def zi(i, _):
        buf_ref[pl.ds(16 * i, 16)] = jnp.full((16,), val, F32)
        return ()
    lax.fori_loop(0, buf_ref.shape[0] // 16, zi, ())


ZF = 512


def _zero_flat_sh(sh_ref, zflat_ref):
    n = sh_ref.shape[0]
    def cp(j, _):
        pltpu.sync_copy(zflat_ref, sh_ref.at[pl.ds(j * ZF, ZF)])
        return ()
    lax.fori_loop(0, n // ZF, cp, ())


def _zero_rows(sh_ref, row0, nrows, zbuf_ref):
    zr = zbuf_ref.shape[0]
    def cp(j, _):
        pltpu.sync_copy(zbuf_ref, sh_ref.at[pl.ds(row0 + j * zr, zr)])
        return ()
    lax.fori_loop(0, nrows // zr, cp, ())


def _rsqrt_nr(d2v):
    i0 = lax.bitcast_convert_type(d2v, I32)
    y = lax.bitcast_convert_type(0x5F3759DF - (i0 >> 1), F32)
    for _ in range(3):
        y = y * (1.5 - 0.5 * d2v * y * y)
    return y


def _bcast_lane(vec, e):
    return vec.at[jnp.full((16,), e, I32)].get(mode="promise_in_bounds")


def _edge_body(ns, ept, u_hbm, v_hbm, px_hbm, py_hbm, pz_hbm, src_hbm,
               dst_hbm, wd_hbm, s_out, d_out, s_sh, d_sh,
               u_b, v_b, si_b, di_b, pxs, pys, pzs, pxd, pyd, pzd,
               wd_b, ones_b, zbuf, zflat):
    cid = lax.axis_index("c")
    sid = lax.axis_index("s")
    wid = _wid()
    nr = ns // NSUB

    _fill_rows(zbuf, 0.0)
    _zero_rows(s_sh, sid * nr, nr, zbuf)
    _fill_flat(ones_b, 1.0)
    @pl.when(sid == 0)
    def _():
        _fill_flat(zflat, 0.0)
        _zero_flat_sh(d_sh, zflat)
    pltpu.sync_copy(wd_hbm, wd_b)
    plsc.subcore_barrier()

    wdv = [wd_b[pl.ds(16 * k, 16)] for k in range(8)]
    base = wid * ept

    def chunk(jc, _):
        off = base + jc * ECHUNK
        pltpu.sync_copy(src_hbm.at[pl.ds(off, ECHUNK)], si_b)
        pltpu.sync_copy(dst_hbm.at[pl.ds(off, ECHUNK)], di_b)
        pltpu.sync_copy(u_hbm.at[si_b], u_b)
        pltpu.sync_copy(v_hbm.at[di_b], v_b)
        pltpu.sync_copy(px_hbm.at[si_b], pxs)
        pltpu.sync_copy(py_hbm.at[si_b], pys)
        pltpu.sync_copy(pz_hbm.at[si_b], pzs)
        pltpu.sync_copy(px_hbm.at[di_b], pxd)
        pltpu.sync_copy(py_hbm.at[di_b], pyd)
        pltpu.sync_copy(pz_hbm.at[di_b], pzd)

        def group(g, _):
            gb = g * 16
            cs16 = pl.ds(gb, 16)
            dx = pxs[cs16] - pxd[cs16]
            dy = pys[cs16] - pyd[cs16]
            dz = pzs[cs16] - pzd[cs16]
            d2v = dx * dx + dy * dy + dz * dz + 1e-12
            dist = d2v * _rsqrt_nr(d2v)
            for e in range(16):
                row = gb + e
                db = _bcast_lane(dist, e)
                for k in range(8):
                    cs = pl.ds(16 * k, 16)
                    pre = u_b[row, cs] + v_b[row, cs] + db * wdv[k]
                    u_b[row, cs] = jnp.maximum(pre, 0.0)
            return ()
        lax.fori_loop(0, ECHUNK // 16, group, ())
        pltpu.sync_copy(u_b, s_sh.at[di_b], add=True)
        pltpu.sync_copy(ones_b, d_sh.at[di_b], add=True)
        return ()
    lax.fori_loop(0, ept // ECHUNK, chunk, ())
    plsc.subcore_barrier()

    r0 = sid * nr
    pltpu.sync_copy(s_sh.at[pl.ds(r0, nr)], s_out.at[cid, pl.ds(r0, nr)])
    @pl.when(sid == 0)
    def _():
        pltpu.sync_copy(d_sh, d_out.at[pl.ds(cid * ns, ns)])


def _sc_edge(u, v, pxyz, src_pad, dst_pad, wd):
    ns = u.shape[0]
    ept = src_pad.shape[0] // NW
    f = pl.kernel(
        functools.partial(_edge_body, ns, ept),
        out_type=[
            jax.ShapeDtypeStruct((NCORE, ns, LD), F32),
            jax.ShapeDtypeStruct((NCORE * ns,), F32),
        ],
        mesh=_sc_mesh(),
        scratch_types=[
            pltpu.VMEM_SHARED((ns, LD), F32),
            pltpu.VMEM_SHARED((ns,), F32),
            pltpu.VMEM((ECHUNK, LD), F32),
            pltpu.VMEM((ECHUNK, LD), F32),
            pltpu.VMEM((ECHUNK,), I32),
            pltpu.VMEM((ECHUNK,), I32),
            pltpu.VMEM((ECHUNK,), F32),
            pltpu.VMEM((ECHUNK,), F32),
            pltpu.VMEM((ECHUNK,), F32),
            pltpu.VMEM((ECHUNK,), F32),
            pltpu.VMEM((ECHUNK,), F32),
            pltpu.VMEM((ECHUNK,), F32),
            pltpu.VMEM((LD,), F32),
            pltpu.VMEM((ECHUNK,), F32),
            pltpu.VMEM((16, LD), F32),
            pltpu.VMEM((ZF,), F32),
        ],
    )
    s_part, d_flat = f(u, v, pxyz[0], pxyz[1], pxyz[2], src_pad, dst_pad, wd)
    return s_part, d_flat[:ns], d_flat[ns:]


def _prep1_body(ns, ept, src_hbm, d_out, d_sh, si_b, ones_b, zflat):
    cid = lax.axis_index("c")
    sid = lax.axis_index("s")
    wid = _wid()
    _fill_flat(ones_b, 1.0)
    @pl.when(sid == 0)
    def _():
        _fill_flat(zflat, 0.0)
        _zero_flat_sh(d_sh, zflat)
    plsc.subcore_barrier()
    base = wid * ept

    def chunk(jc, _):
        pltpu.sync_copy(src_hbm.at[pl.ds(base + jc * ECHUNK, ECHUNK)], si_b)
        pltpu.sync_copy(ones_b, d_sh.at[si_b], add=True)
        return ()
    lax.fori_loop(0, ept // ECHUNK, chunk, ())
    plsc.subcore_barrier()
    @pl.when(sid == 0)
    def _():
        pltpu.sync_copy(d_sh, d_out.at[pl.ds(cid * ns, ns)])


def _sc_prep1(src_pad, ns):
    ept = src_pad.shape[0] // NW
    f = pl.kernel(
        functools.partial(_prep1_body, ns, ept),
        out_type=jax.ShapeDtypeStruct((NCORE * ns,), F32),
        mesh=_sc_mesh(),
        scratch_types=[
            pltpu.VMEM_SHARED((ns,), F32),
            pltpu.VMEM((ECHUNK,), I32),
            pltpu.VMEM((ECHUNK,), F32),
            pltpu.VMEM((ZF,), F32),
        ],
    )
    d_flat = f(src_pad)
    return d_flat[:ns], d_flat[ns:]


def _prep2_body(ns, ept, w_hbm, d0_hbm, d1_hbm, src_hbm, dst_hbm,
                a_out, a_sh, si_b, di_b, wg, g0, g1, wts_b, zflat):
    cid = lax.axis_index("c")
    sid = lax.axis_index("s")
    wid = _wid()
    @pl.when(sid == 0)
    def _():
        _fill_flat(zflat, 0.0)
        _zero_flat_sh(a_sh, zflat)
    plsc.subcore_barrier()
    base = wid * ept

    def chunk(jc, _):
        off = base + jc * ECHUNK
        pltpu.sync_copy(src_hbm.at[pl.ds(off, ECHUNK)], si_b)
        pltpu.sync_copy(dst_hbm.at[pl.ds(off, ECHUNK)], di_b)
        pltpu.sync_copy(w_hbm.at[si_b], wg)
        pltpu.sync_copy(d0_hbm.at[si_b], g0)
        pltpu.sync_copy(d1_hbm.at[si_b], g1)

        def group(g, _):
            cs = pl.ds(g * 16, 16)
            deg = jnp.maximum(g0[cs] + g1[cs], 1.0)
            wts_b[cs] = wg[cs] / deg
            return ()
        lax.fori_loop(0, ECHUNK // 16, group, ())
        pltpu.sync_copy(wts_b, a_sh.at[di_b], add=True)
        return ()
    lax.fori_loop(0, ept // ECHUNK, chunk, ())
    plsc.subcore_barrier()
    @pl.when(sid == 0)
    def _():
        pltpu.sync_copy(a_sh, a_out.at[pl.ds(cid * ns, ns)])


def _sc_prep2(w, d0, d1, src_pad, dst_pad):
    ns = w.shape[0]
    ept = src_pad.shape[0] // NW
    f = pl.kernel(
        functools.partial(_prep2_body, ns, ept),
        out_type=jax.ShapeDtypeStruct((NCORE * ns,), F32),
        mesh=_sc_mesh(),
        scratch_types=[
            pltpu.VMEM_SHARED((ns,), F32),
            pltpu.VMEM((ECHUNK,), I32),
            pltpu.VMEM((ECHUNK,), I32),
            pltpu.VMEM((ECHUNK,), F32),
            pltpu.VMEM((ECHUNK,), F32),
            pltpu.VMEM((ECHUNK,), F32),
            pltpu.VMEM((ECHUNK,), F32),
            pltpu.VMEM((ZF,), F32),
        ],
    )
    a_flat = f(w, d0, d1, src_pad, dst_pad)
    return a_flat[:ns], a_flat[ns:]


def _conv_body(ns, ept, with_pos, rows_by_dst, x_hbm, px_hbm, py_hbm, pz_hbm,
               w_hbm, d0_hbm, d1_hbm, a0_hbm, a1_hbm, src_hbm, dst_hbm,
               h_out, px_o, py_o, pz_o,
               h_sh, px_sh, py_sh, pz_sh, x_b, gi_b, sc_b, si_b, di_b,
               wg, g0, g1, a0g, a1g, ec_b, pxg, pyg, pzg, zbuf, zflat):
    cid = lax.axis_index("c")
    sid = lax.axis_index("s")
    wid = _wid()
    nr = ns // NSUB

    _fill_rows(zbuf, 0.0)
    _zero_rows(h_sh, sid * nr, nr, zbuf)
    @pl.when(sid == 0)
    def _():
        _fill_flat(zflat, 0.0)
        if with_pos:
            _zero_flat_sh(px_sh, zflat)
            _zero_flat_sh(py_sh, zflat)
            _zero_flat_sh(pz_sh, zflat)
    plsc.subcore_barrier()
    base = wid * ept

    def chunk(jc, _):
        off = base + jc * ECHUNK
        pltpu.sync_copy(src_hbm.at[pl.ds(off, ECHUNK)], si_b)
        pltpu.sync_copy(dst_hbm.at[pl.ds(off, ECHUNK)], di_b)
        gi = di_b if rows_by_dst else si_b
        sc = si_b if rows_by_dst else di_b
        pltpu.sync_copy(x_hbm.at[gi], x_b)
        pltpu.sync_copy(w_hbm.at[si_b], wg)
        pltpu.sync_copy(d0_hbm.at[si_b], g0)
        pltpu.sync_copy(d1_hbm.at[si_b], g1)
        pltpu.sync_copy(a0_hbm.at[di_b], a0g)
        pltpu.sync_copy(a1_hbm.at[di_b], a1g)
        if with_pos:
            pltpu.sync_copy(px_hbm.at[si_b], pxg)
            pltpu.sync_copy(py_hbm.at[si_b], pyg)
            pltpu.sync_copy(pz_hbm.at[si_b], pzg)

        def group(g, _):
            gb = g * 16
            cs16 = pl.ds(gb, 16)
            deg = jnp.maximum(g0[cs16] + g1[cs16], 1.0)
            ecv = (wg[cs16] / deg) / (a0g[cs16] + a1g[cs16] + 1e-12)
            ec_b[cs16] = ecv
            if with_pos:
                pxg[cs16] = ecv * pxg[cs16]
                pyg[cs16] = ecv * pyg[cs16]
                pzg[cs16] = ecv * pzg[cs16]
            for e in range(16):
                row = gb + e
                eb = _bcast_lane(ecv, e)
                for k in range(8):
                    cs = pl.ds(16 * k, 16)
                    x_b[row, cs] = eb * x_b[row, cs]
            return ()
        lax.fori_loop(0, ECHUNK // 16, group, ())
        pltpu.sync_copy(x_b, h_sh.at[sc], add=True)
        if with_pos:
            pltpu.sync_copy(pxg, px_sh.at[sc], add=True)
            pltpu.sync_copy(pyg, py_sh.at[sc], add=True)
            pltpu.sync_copy(pzg, pz_sh.at[sc], add=True)
        return ()
    lax.fori_loop(0, ept // ECHUNK, chunk, ())
    plsc.subcore_barrier()

    r0 = sid * nr
    pltpu.sync_copy(h_sh.at[pl.ds(r0, nr)], h_out.at[cid, pl.ds(r0, nr)])
    if with_pos:
        @pl.when(sid == 0)
        def _():
            pltpu.sync_copy(px_sh, px_o.at[pl.ds(cid * ns, ns)])
            pltpu.sync_copy(py_sh, py_o.at[pl.ds(cid * ns, ns)])
            pltpu.sync_copy(pz_sh, pz_o.at[pl.ds(cid * ns, ns)])


def _sc_conv(x, pxyz, w, d0, d1, a0, a1, src_pad, dst_pad, with_pos,
             rows_by_dst):
    ns = x.shape[0]
    ept = src_pad.shape[0] // NW
    out_type = [jax.ShapeDtypeStruct((NCORE, ns, LD), F32)]
    out_type += [jax.ShapeDtypeStruct((NCORE * ns,), F32)] * 3
    scr = [
        pltpu.VMEM_SHARED((ns, LD), F32),
        pltpu.VMEM_SHARED((ns,), F32),
        pltpu.VMEM_SHARED((ns,), F32),
        pltpu.VMEM_SHARED((ns,), F32),
        pltpu.VMEM((ECHUNK, LD), F32),
        pltpu.VMEM((ECHUNK,), I32),
        pltpu.VMEM((ECHUNK,), I32),
        pltpu.VMEM((ECHUNK,), I32),
        pltpu.VMEM((ECHUNK,), I32),
        pltpu.VMEM((ECHUNK,), F32),
        pltpu.VMEM((ECHUNK,), F32),
        pltpu.VMEM((ECHUNK,), F32),
        pltpu.VMEM((ECHUNK,), F32),
        pltpu.VMEM((ECHUNK,), F32),
        pltpu.VMEM((ECHUNK,), F32),
        pltpu.VMEM((ECHUNK,), F32),
        pltpu.VMEM((ECHUNK,), F32),
        pltpu.VMEM((ECHUNK,), F32),
        pltpu.VMEM((16, LD), F32),
        pltpu.VMEM((ZF,), F32),
    ]
    f = pl.kernel(
        functools.partial(_conv_body, ns, ept, with_pos, rows_by_dst),
        out_type=out_type,
        mesh=_sc_mesh(),
        scratch_types=scr,
    )
    px, py, pz = pxyz if with_pos else (w, w, w)
    return f(x, px, py, pz, w, d0, d1, a0, a1, src_pad, dst_pad)


GCH = 80


def _compact_body(ns_out, hp0_hbm, hp1_hbm, px0, px1, py0, py1, pz0, pz1,
                  aw0, aw1, mid_hbm, h_out, px_o, py_o, pz_o, w_o,
                  mid_b, r0_b, r1_b, s0_b, s1_b):
    wid = _wid()
    nrw = ns_out // NW
    base = wid * nrw
    for j0 in range(0, nrw, GCH):
        off = base + j0
        pltpu.sync_copy(mid_hbm.at[pl.ds(off, GCH)], mid_b)
        pltpu.sync_copy(hp0_hbm.at[mid_b], r0_b)
        pltpu.sync_copy(hp1_hbm.at[mid_b], r1_b)

        def add_rows(i, _):
            for k in range(8):
                cs = pl.ds(16 * k, 16)
                r0_b[i, cs] = r0_b[i, cs] + r1_b[i, cs]
            return ()
        lax.fori_loop(0, GCH, add_rows, ())
        pltpu.sync_copy(r0_b, h_out.at[pl.ds(off, GCH)])

        for (a_, b_, o_, eps) in ((px0, px1, px_o, 0.0),
                                  (py0, py1, py_o, 0.0),
                                  (pz0, pz1, pz_o, 0.0),
                                  (aw0, aw1, w_o, 1e-12)):
            pltpu.sync_copy(a_.at[mid_b], s0_b)
            pltpu.sync_copy(b_.at[mid_b], s1_b)

            def add_s(i, _):
                cs = pl.ds(16 * i, 16)
                s0_b[cs] = s0_b[cs] + s1_b[cs] + eps
                return ()
            lax.fori_loop(0, GCH // 16, add_s, ())
            pltpu.sync_copy(s0_b, o_.at[pl.ds(off, GCH)])


def _sc_compact(hp, pxp, pyp, pzp, awp, mid_pad):
    ns_out = mid_pad.shape[0]
    f = pl.kernel(
        functools.partial(_compact_body, ns_out),
        out_type=[
            jax.ShapeDtypeStruct((ns_out, LD), F32),
            jax.ShapeDtypeStruct((ns_out,), F32),
            jax.ShapeDtypeStruct((ns_out,), F32),
            jax.ShapeDtypeStruct((ns_out,), F32),
            jax.ShapeDtypeStruct((ns_out,), F32),
        ],
        mesh=_sc_mesh(),
        scratch_types=[
            pltpu.VMEM((GCH,), I32),
            pltpu.VMEM((GCH, LD), F32),
            pltpu.VMEM((GCH, LD), F32),
            pltpu.VMEM((GCH,), F32),
            pltpu.VMEM((GCH,), F32),
        ],
    )
    return f(hp[0], hp[1], pxp[0], pxp[1], pyp[0], pyp[1], pzp[0], pzp[1],
             awp[0], awp[1], mid_pad)


def _unpool_body(ns_out, hse_hbm, sel_hbm, u_out, sel_b, r_b):
    wid = _wid()
    nrw = ns_out // NW
    base = wid * nrw
    for j0 in range(0, nrw, GCH):
        off = base + j0
        pltpu.sync_copy(sel_hbm.at[pl.ds(off, GCH)], sel_b)
        pltpu.sync_copy(hse_hbm.at[sel_b], r_b)
        pltpu.sync_copy(r_b, u_out.at[pl.ds(off, GCH)])


def _sc_unpool(h_small_ext, sel_pad):
    ns_out = sel_pad.shape[0]
    f = pl.kernel(
        functools.partial(_unpool_body, ns_out),
        out_type=jax.ShapeDtypeStruct((ns_out, LD), F32),
        mesh=_sc_mesh(),
        scratch_types=[
            pltpu.VMEM((GCH,), I32),
            pltpu.VMEM((GCH, LD), F32),
        ],
    )
    return f(h_small_ext, sel_pad)


def _pre_body(h_ref, pos_ref, wa_ref, wb_ref, wp_ref, b1_ref, u_ref, v_ref):
    h = h_ref[...]
    posp = pos_ref[...] @ wp_ref[...]
    u_ref[...] = h @ wa_ref[...] + posp + b1_ref[...]
    v_ref[...] = h @ wb_ref[...] - posp


def _tc_pre(h, posr, wa, wb, wp, b1, blk=512):
    n = h.shape[0]
    return pl.pallas_call(
        _pre_body,
        grid=(_ceil(n, blk),),
        in_specs=[
            pl.BlockSpec((blk, LD), lambda i: (i, 0)),
            pl.BlockSpec((blk, 16), lambda i: (i, 0)),
            pl.BlockSpec((LD, LD), lambda i: (0, 0)),
            pl.BlockSpec((LD, LD), lambda i: (0, 0)),
            pl.BlockSpec((16, LD), lambda i: (0, 0)),
            pl.BlockSpec((1, LD), lambda i: (0, 0)),
        ],
        out_specs=[
            pl.BlockSpec((blk, LD), lambda i: (i, 0)),
            pl.BlockSpec((blk, LD), lambda i: (i, 0)),
        ],
        out_shape=[
            jax.ShapeDtypeStruct((n, LD), F32),
            jax.ShapeDtypeStruct((n, LD), F32),
        ],
    )(h, posr, wa, wb, wp, b1)


def _node_body(h_ref, s0_ref, s1_ref, d0_ref, d1_ref, w2e_ref, b2e_ref,
               wna_ref, wnb_ref, bn1_ref, wn2_ref, bn2_ref, res_ref, o_ref):
    h = h_ref[...]
    deg = d0_ref[...] + d1_ref[...]
    aggr = (s0_ref[0] + s1_ref[0]) @ w2e_ref[...] + deg * b2e_ref[...]
    z = jnp.maximum(h @ wna_ref[...] + aggr @ wnb_ref[...] + bn1_ref[...],
                    0.0)
    o_ref[...] = h + z @ wn2_ref[...] + bn2_ref[...] + res_ref[...]


def _tc_node(h, s_part, d0, d1, w2e, b2e, wna, wnb, bn1, wn2, bn2, res,
             blk=512):
    n = h.shape[0]
    return pl.pallas_call(
        _node_body,
        grid=(_ceil(n, blk),),
        in_specs=[
            pl.BlockSpec((blk, LD), lambda i: (i, 0)),
            pl.BlockSpec((1, blk, LD), lambda i: (0, i, 0)),
            pl.BlockSpec((1, blk, LD), lambda i: (1, i, 0)),
            pl.BlockSpec((blk, 1), lambda i: (i, 0)),
            pl.BlockSpec((blk, 1), lambda i: (i, 0)),
            pl.BlockSpec((LD, LD), lambda i: (0, 0)),
            pl.BlockSpec((1, LD), lambda i: (0, 0)),
            pl.BlockSpec((LD, LD), lambda i: (0, 0)),
            pl.BlockSpec((LD, LD), lambda i: (0, 0)),
            pl.BlockSpec((1, LD), lambda i: (0, 0)),
            pl.BlockSpec((LD, LD), lambda i: (0, 0)),
            pl.BlockSpec((1, LD), lambda i: (0, 0)),
            pl.BlockSpec((blk, LD), lambda i: (i, 0)),
        ],
        out_specs=pl.BlockSpec((blk, LD), lambda i: (i, 0)),
        out_shape=jax.ShapeDtypeStruct((n, LD), F32),
    )(h, s_part, s_part, d0[:, None], d1[:, None], w2e, b2e, wna, wnb, bn1,
      wn2, bn2, res)


def _add_body(a_ref, b_ref, o_ref):
    o_ref[...] = a_ref[0] + b_ref[0]


def _tc_add2(parts, blk=512):
    n = parts.shape[1]
    return pl.pallas_call(
        _add_body,
        grid=(_ceil(n, blk),),
        in_specs=[
            pl.BlockSpec((1, blk, LD), lambda i: (0, i, 0)),
            pl.BlockSpec((1, blk, LD), lambda i: (1, i, 0)),
        ],
        out_specs=pl.BlockSpec((blk, LD), lambda i: (i, 0)),
        out_shape=jax.ShapeDtypeStruct((n, LD), F32),
    )(parts, parts)


def _split_gmp_params(p):
    w1e = p["edge"]["W1"]
    wa = w1e[:LD]
    wb = w1e[LD:2 * LD]
    wp = jnp.pad(w1e[2 * LD:2 * LD + 3], ((0, 13), (0, 0)))
    wd = w1e[2 * LD + 3]
    b1 = p["edge"]["b1"][None, :]
    w2e = p["edge"]["W2"]
    b2e = p["edge"]["b2"][None, :]
    wn1 = p["node"]["W1"]
    return (wa, wb, wp, wd, b1, w2e, b2e, wn1[:LD], wn1[LD:],
            p["node"]["b1"][None, :], p["node"]["W2"],
            p["node"]["b2"][None, :])


def _pad_edges(src, dst, n):
    e = src.shape[0]
    epad = _ceil(e, NW * ECHUNK) * NW * ECHUNK
    fill = jnp.full((epad - e,), n, I32)
    return jnp.concatenate([src, fill]), jnp.concatenate([dst, fill])


def _gmp_sc(p, h_pad, posr, pxyz, srcp, dstp, res=None):
    (wa, wb, wp, wd, b1, w2e, b2e, wna, wnb, bn1, wn2, bn2) = \
        _split_gmp_params(p)
    u, v = _tc_pre(h_pad, posr, wa, wb, wp, b1)
    s_part, dd0, dd1 = _sc_edge(u, v, pxyz, srcp, dstp, wd)
    if res is None:
        res = jnp.zeros_like(h_pad)
    return _tc_node(h_pad, s_part, dd0, dd1, w2e, b2e, wna, wnb, bn1, wn2,
                    bn2, res)


def _pad_rows(x, ns):
    return jnp.pad(x, ((0, ns - x.shape[0]), (0, 0)))


def _pad_flat(x, ns):
    return jnp.pad(x, (0, ns - x.shape[0]))


def kernel(h, pos, params, m_gs_0, m_gs_1, m_gs_2, m_ids_0, m_ids_1):
    m_gs = [m_gs_0, m_gs_1, m_gs_2]
    m_ids = [m_ids_0, m_ids_1]
    nlist = [10000, 5000, 2500]

    ns0 = NS_TAB[10000]
    h_pad = _pad_rows(h, ns0)
    posr = jnp.pad(pos, ((0, ns0 - pos.shape[0]), (0, 13)))
    pxyz = (posr[:, 0], posr[:, 1], posr[:, 2])
    w = _pad_flat(jnp.ones((10000,), F32), ns0)

    down_hs, down_geo, down_sc, pads = [], [], [], []
    for i in range(2):
        n = nlist[i]
        ns = NS_TAB[n]
        nsn = NS_TAB[nlist[i + 1]]
        srcp, dstp = _pad_edges(m_gs[i][0], m_gs[i][1], n)
        pads.append((srcp, dstp))
        h_pad = _gmp_sc(params["down"][i], h_pad, posr, pxyz, srcp, dstp)
        down_hs.append(h_pad)
        down_geo.append((posr, pxyz))
        d0, d1 = _sc_prep1(srcp, ns)
        a0, a1 = _sc_prep2(w, d0, d1, srcp, dstp)
        down_sc.append((w, d0, d1, a0, a1))
        hp, pxf, pyf, pzf = _sc_conv(
            h_pad, pxyz, w, d0, d1, a0, a1, srcp, dstp, True, False)
        mid_pad = _pad_flat(m_ids[i], nsn)
        h_pad, px, py, pz, w = _sc_compact(
            hp, (pxf[:ns], pxf[ns:]), (pyf[:ns], pyf[ns:]),
            (pzf[:ns], pzf[ns:]), (a0, a1), mid_pad)
        posr = jnp.concatenate(
            [px[:, None], py[:, None], pz[:, None],
             jnp.zeros((nsn, 13), F32)], axis=1)
        pxyz = (px, py, pz)

    n2 = nlist[2]
    ns2 = NS_TAB[n2]
    srcp, dstp = _pad_edges(m_gs[2][0], m_gs[2][1], n2)
    h_pad = _gmp_sc(params["bottom"], h_pad, posr, pxyz, srcp, dstp)

    for i in range(2):
        li = 1 - i
        n = nlist[li]
        ns = NS_TAB[n]
        ns_small = NS_TAB[nlist[li + 1]]
        srcp, dstp = pads[li]
        posr, pxyz = down_geo[li]
        w_l, d0, d1, a0, a1 = down_sc[li]
        hse = jnp.concatenate([h_pad, jnp.zeros((LD, LD), F32)], axis=0)
        zrow = ns_small
        sel = jnp.full((ns,), zrow, I32).at[m_ids[li]].set(
            jnp.arange(nlist[li + 1], dtype=I32))
        u_arr = _sc_unpool(hse, sel)
        hu = _sc_conv(u_arr, None, w_l, d0, d1, a0, a1, srcp, dstp,
                      False, True)[0]
        h_uc = _tc_add2(hu)
        h_pad = _gmp_sc(params["up"][i], h_uc, posr, pxyz, srcp, dstp,
                        res=down_hs[li])
    return h_pad[:10000]


# speedup vs baseline: 3.1791x; 1.5893x over previous
import functools
import jax
import jax.numpy as jnp
from jax import lax
from jax.experimental import pallas as pl
from jax.experimental.pallas import tpu as pltpu
from jax.experimental.pallas import tpu_sc as plsc

LD = 128
NCORE = 2
NSUB = 16
NW = NCORE * NSUB
ECHUNK = 128
F32 = jnp.float32
I32 = jnp.int32

NS_TAB = {10000: 10240, 5000: 5120, 2500: 2560}


def _ceil(a, b):
    return (a + b - 1) // b


def _sc_mesh():
    return plsc.VectorSubcoreMesh(core_axis_name="c", subcore_axis_name="s",
                                  num_cores=NCORE, num_subcores=NSUB)


def _wid():
    return lax.axis_index("s") * NCORE + lax.axis_index("c")


def _fill_rows(buf_ref, val):
    def zrow(i, _):
        for k in range(buf_ref.shape[1] // 16):
            buf_ref[i, pl.ds(16 * k, 16)] = jnp.full((16,), val, F32)
        return ()
    lax.fori_loop(0, buf_ref.shape[0], zrow, ())


def _fill_flat(buf_ref, val):
    def zi(i, _):
        buf_ref[pl.ds(16 * i, 16)] = jnp.full((16,), val, F32)
        return ()
    lax.fori_loop(0, buf_ref.shape[0] // 16, zi, ())


ZF = 512


def _zero_flat_sh(sh_ref, zflat_ref):
    n = sh_ref.shape[0]
    def cp(j, _):
        pltpu.sync_copy(zflat_ref, sh_ref.at[pl.ds(j * ZF, ZF)])
        return ()
    lax.fori_loop(0, n // ZF, cp, ())


def _zero_rows(sh_ref, row0, nrows, zbuf_ref):
    zr = zbuf_ref.shape[0]
    def cp(j, _):
        pltpu.sync_copy(zbuf_ref, sh_ref.at[pl.ds(row0 + j * zr, zr)])
        return ()
    lax.fori_loop(0, nrows // zr, cp, ())


def _rsqrt_nr(d2v):
    i0 = lax.bitcast_convert_type(d2v, I32)
    y = lax.bitcast_convert_type(0x5F3759DF - (i0 >> 1), F32)
    for _ in range(3):
        y = y * (1.5 - 0.5 * d2v * y * y)
    return y


def _bcast_lane(vec, e):
    return vec.at[jnp.full((16,), e, I32)].get(mode="promise_in_bounds")


def _edge_body(ns, ept, u_hbm, v_hbm, px_hbm, py_hbm, pz_hbm, src_hbm,
               dst_hbm, wd_hbm, s_out, d_out, s_sh, d_sh,
               u_b, v_b, si_b, di_b, pxs, pys, pzs, pxd, pyd, pzd,
               wd_b, ones_b, zbuf, zflat, sem):
    cid = lax.axis_index("c")
    sid = lax.axis_index("s")
    wid = _wid()
    nr = ns // NSUB

    _fill_rows(zbuf, 0.0)
    _zero_rows(s_sh, sid * nr, nr, zbuf)
    _fill_flat(ones_b, 1.0)
    @pl.when(sid == 0)
    def _():
        _fill_flat(zflat, 0.0)
        _zero_flat_sh(d_sh, zflat)
    pltpu.sync_copy(wd_hbm, wd_b)
    plsc.subcore_barrier()

    wdv = [wd_b[pl.ds(16 * k, 16)] for k in range(8)]
    base = wid * ept

    def chunk(jc, _):
        off = base + jc * ECHUNK
        d1_ = pltpu.async_copy(src_hbm.at[pl.ds(off, ECHUNK)], si_b, sem)
        d2_ = pltpu.async_copy(dst_hbm.at[pl.ds(off, ECHUNK)], di_b, sem)
        d1_.wait()
        d2_.wait()
        descs = [
            pltpu.async_copy(u_hbm.at[si_b], u_b, sem),
            pltpu.async_copy(v_hbm.at[di_b], v_b, sem),
            pltpu.async_copy(px_hbm.at[si_b], pxs, sem),
            pltpu.async_copy(py_hbm.at[si_b], pys, sem),
            pltpu.async_copy(pz_hbm.at[si_b], pzs, sem),
            pltpu.async_copy(px_hbm.at[di_b], pxd, sem),
            pltpu.async_copy(py_hbm.at[di_b], pyd, sem),
            pltpu.async_copy(pz_hbm.at[di_b], pzd, sem),
        ]
        for d_ in descs:
            d_.wait()

        def group(g, _):
            gb = g * 16
            cs16 = pl.ds(gb, 16)
            dx = pxs[cs16] - pxd[cs16]
            dy = pys[cs16] - pyd[cs16]
            dz = pzs[cs16] - pzd[cs16]
            d2v = dx * dx + dy * dy + dz * dz + 1e-12
            dist = d2v * _rsqrt_nr(d2v)
            for e in range(16):
                row = gb + e
                db = _bcast_lane(dist, e)
                for k in range(8):
                    cs = pl.ds(16 * k, 16)
                    pre = u_b[row, cs] + v_b[row, cs] + db * wdv[k]
                    u_b[row, cs] = jnp.maximum(pre, 0.0)
            return ()
        lax.fori_loop(0, ECHUNK // 16, group, ())
        d3_ = pltpu.async_copy(u_b, s_sh.at[di_b], sem, add=True)
        d4_ = pltpu.async_copy(ones_b, d_sh.at[di_b], sem, add=True)
        d3_.wait()
        d4_.wait()
        return ()
    lax.fori_loop(0, ept // ECHUNK, chunk, ())
    plsc.subcore_barrier()

    r0 = sid * nr
    pltpu.sync_copy(s_sh.at[pl.ds(r0, nr)], s_out.at[cid, pl.ds(r0, nr)])
    @pl.when(sid == 0)
    def _():
        pltpu.sync_copy(d_sh, d_out.at[pl.ds(cid * ns, ns)])


def _sc_edge(u, v, pxyz, src_pad, dst_pad, wd):
    ns = u.shape[0]
    ept = src_pad.shape[0] // NW
    f = pl.kernel(
        functools.partial(_edge_body, ns, ept),
        out_type=[
            jax.ShapeDtypeStruct((NCORE, ns, LD), F32),
            jax.ShapeDtypeStruct((NCORE * ns,), F32),
        ],
        mesh=_sc_mesh(),
        scratch_types=[
            pltpu.VMEM_SHARED((ns, LD), F32),
            pltpu.VMEM_SHARED((ns,), F32),
            pltpu.VMEM((ECHUNK, LD), F32),
            pltpu.VMEM((ECHUNK, LD), F32),
            pltpu.VMEM((ECHUNK,), I32),
            pltpu.VMEM((ECHUNK,), I32),
            pltpu.VMEM((ECHUNK,), F32),
            pltpu.VMEM((ECHUNK,), F32),
            pltpu.VMEM((ECHUNK,), F32),
            pltpu.VMEM((ECHUNK,), F32),
            pltpu.VMEM((ECHUNK,), F32),
            pltpu.VMEM((ECHUNK,), F32),
            pltpu.VMEM((LD,), F32),
            pltpu.VMEM((ECHUNK,), F32),
            pltpu.VMEM((16, LD), F32),
            pltpu.VMEM((ZF,), F32),
            pltpu.SemaphoreType.DMA,
        ],
    )
    s_part, d_flat = f(u, v, pxyz[0], pxyz[1], pxyz[2], src_pad, dst_pad, wd)
    return s_part, d_flat[:ns], d_flat[ns:]


def _prep1_body(ns, ept, src_hbm, d_out, d_sh, si_b, ones_b, zflat):
    cid = lax.axis_index("c")
    sid = lax.axis_index("s")
    wid = _wid()
    _fill_flat(ones_b, 1.0)
    @pl.when(sid == 0)
    def _():
        _fill_flat(zflat, 0.0)
        _zero_flat_sh(d_sh, zflat)
    plsc.subcore_barrier()
    base = wid * ept

    def chunk(jc, _):
        pltpu.sync_copy(src_hbm.at[pl.ds(base + jc * ECHUNK, ECHUNK)], si_b)
        pltpu.sync_copy(ones_b, d_sh.at[si_b], add=True)
        return ()
    lax.fori_loop(0, ept // ECHUNK, chunk, ())
    plsc.subcore_barrier()
    @pl.when(sid == 0)
    def _():
        pltpu.sync_copy(d_sh, d_out.at[pl.ds(cid * ns, ns)])


def _sc_prep1(src_pad, ns):
    ept = src_pad.shape[0] // NW
    f = pl.kernel(
        functools.partial(_prep1_body, ns, ept),
        out_type=jax.ShapeDtypeStruct((NCORE * ns,), F32),
        mesh=_sc_mesh(),
        scratch_types=[
            pltpu.VMEM_SHARED((ns,), F32),
            pltpu.VMEM((ECHUNK,), I32),
            pltpu.VMEM((ECHUNK,), F32),
            pltpu.VMEM((ZF,), F32),
        ],
    )
    d_flat = f(src_pad)
    return d_flat[:ns], d_flat[ns:]


def _prep2_body(ns, ept, w_hbm, d0_hbm, d1_hbm, src_hbm, dst_hbm,
                a_out, a_sh, si_b, di_b, wg, g0, g1, wts_b, zflat, sem):
    cid = lax.axis_index("c")
    sid = lax.axis_index("s")
    wid = _wid()
    @pl.when(sid == 0)
    def _():
        _fill_flat(zflat, 0.0)
        _zero_flat_sh(a_sh, zflat)
    plsc.subcore_barrier()
    base = wid * ept

    def chunk(jc, _):
        off = base + jc * ECHUNK
        d1_ = pltpu.async_copy(src_hbm.at[pl.ds(off, ECHUNK)], si_b, sem)
        d2_ = pltpu.async_copy(dst_hbm.at[pl.ds(off, ECHUNK)], di_b, sem)
        d1_.wait()
        d2_.wait()
        descs = [
            pltpu.async_copy(w_hbm.at[si_b], wg, sem),
            pltpu.async_copy(d0_hbm.at[si_b], g0, sem),
            pltpu.async_copy(d1_hbm.at[si_b], g1, sem),
        ]
        for d_ in descs:
            d_.wait()

        def group(g, _):
            cs = pl.ds(g * 16, 16)
            deg = jnp.maximum(g0[cs] + g1[cs], 1.0)
            wts_b[cs] = wg[cs] / deg
            return ()
        lax.fori_loop(0, ECHUNK // 16, group, ())
        pltpu.sync_copy(wts_b, a_sh.at[di_b], add=True)
        return ()
    lax.fori_loop(0, ept // ECHUNK, chunk, ())
    plsc.subcore_barrier()
    @pl.when(sid == 0)
    def _():
        pltpu.sync_copy(a_sh, a_out.at[pl.ds(cid * ns, ns)])


def _sc_prep2(w, d0, d1, src_pad, dst_pad):
    ns = w.shape[0]
    ept = src_pad.shape[0] // NW
    f = pl.kernel(
        functools.partial(_prep2_body, ns, ept),
        out_type=jax.ShapeDtypeStruct((NCORE * ns,), F32),
        mesh=_sc_mesh(),
        scratch_types=[
            pltpu.VMEM_SHARED((ns,), F32),
            pltpu.VMEM((ECHUNK,), I32),
            pltpu.VMEM((ECHUNK,), I32),
            pltpu.VMEM((ECHUNK,), F32),
            pltpu.VMEM((ECHUNK,), F32),
            pltpu.VMEM((ECHUNK,), F32),
            pltpu.VMEM((ECHUNK,), F32),
            pltpu.VMEM((ZF,), F32),
            pltpu.SemaphoreType.DMA,
        ],
    )
    a_flat = f(w, d0, d1, src_pad, dst_pad)
    return a_flat[:ns], a_flat[ns:]


def _conv_body(ns, ept, with_pos, rows_by_dst, x_hbm, px_hbm, py_hbm, pz_hbm,
               w_hbm, d0_hbm, d1_hbm, a0_hbm, a1_hbm, src_hbm, dst_hbm,
               h_out, px_o, py_o, pz_o,
               h_sh, px_sh, py_sh, pz_sh, x_b, gi_b, sc_b, si_b, di_b,
               wg, g0, g1, a0g, a1g, ec_b, pxg, pyg, pzg, zbuf, zflat, sem):
    cid = lax.axis_index("c")
    sid = lax.axis_index("s")
    wid = _wid()
    nr = ns // NSUB

    _fill_rows(zbuf, 0.0)
    _zero_rows(h_sh, sid * nr, nr, zbuf)
    @pl.when(sid == 0)
    def _():
        _fill_flat(zflat, 0.0)
        if with_pos:
            _zero_flat_sh(px_sh, zflat)
            _zero_flat_sh(py_sh, zflat)
            _zero_flat_sh(pz_sh, zflat)
    plsc.subcore_barrier()
    base = wid * ept

    def chunk(jc, _):
        off = base + jc * ECHUNK
        d1_ = pltpu.async_copy(src_hbm.at[pl.ds(off, ECHUNK)], si_b, sem)
        d2_ = pltpu.async_copy(dst_hbm.at[pl.ds(off, ECHUNK)], di_b, sem)
        d1_.wait()
        d2_.wait()
        gi = di_b if rows_by_dst else si_b
        sc = si_b if rows_by_dst else di_b
        descs = [
            pltpu.async_copy(x_hbm.at[gi], x_b, sem),
            pltpu.async_copy(w_hbm.at[si_b], wg, sem),
            pltpu.async_copy(d0_hbm.at[si_b], g0, sem),
            pltpu.async_copy(d1_hbm.at[si_b], g1, sem),
            pltpu.async_copy(a0_hbm.at[di_b], a0g, sem),
            pltpu.async_copy(a1_hbm.at[di_b], a1g, sem),
        ]
        if with_pos:
            descs += [
                pltpu.async_copy(px_hbm.at[si_b], pxg, sem),
                pltpu.async_copy(py_hbm.at[si_b], pyg, sem),
                pltpu.async_copy(pz_hbm.at[si_b], pzg, sem),
            ]
        for d_ in descs:
            d_.wait()

        def group(g, _):
            gb = g * 16
            cs16 = pl.ds(gb, 16)
            deg = jnp.maximum(g0[cs16] + g1[cs16], 1.0)
            ecv = (wg[cs16] / deg) / (a0g[cs16] + a1g[cs16] + 1e-12)
            ec_b[cs16] = ecv
            if with_pos:
                pxg[cs16] = ecv * pxg[cs16]
                pyg[cs16] = ecv * pyg[cs16]
                pzg[cs16] = ecv * pzg[cs16]
            for e in range(16):
                row = gb + e
                eb = _bcast_lane(ecv, e)
                for k in range(8):
                    cs = pl.ds(16 * k, 16)
                    x_b[row, cs] = eb * x_b[row, cs]
            return ()
        lax.fori_loop(0, ECHUNK // 16, group, ())
        descs2 = [pltpu.async_copy(x_b, h_sh.at[sc], sem, add=True)]
        if with_pos:
            descs2 += [
                pltpu.async_copy(pxg, px_sh.at[sc], sem, add=True),
                pltpu.async_copy(pyg, py_sh.at[sc], sem, add=True),
                pltpu.async_copy(pzg, pz_sh.at[sc], sem, add=True),
            ]
        for d_ in descs2:
            d_.wait()
        return ()
    lax.fori_loop(0, ept // ECHUNK, chunk, ())
    plsc.subcore_barrier()

    r0 = sid * nr
    pltpu.sync_copy(h_sh.at[pl.ds(r0, nr)], h_out.at[cid, pl.ds(r0, nr)])
    if with_pos:
        @pl.when(sid == 0)
        def _():
            pltpu.sync_copy(px_sh, px_o.at[pl.ds(cid * ns, ns)])
            pltpu.sync_copy(py_sh, py_o.at[pl.ds(cid * ns, ns)])
            pltpu.sync_copy(pz_sh, pz_o.at[pl.ds(cid * ns, ns)])


def _sc_conv(x, pxyz, w, d0, d1, a0, a1, src_pad, dst_pad, with_pos,
             rows_by_dst):
    ns = x.shape[0]
    ept = src_pad.shape[0] // NW
    out_type = [jax.ShapeDtypeStruct((NCORE, ns, LD), F32)]
    out_type += [jax.ShapeDtypeStruct((NCORE * ns,), F32)] * 3
    scr = [
        pltpu.VMEM_SHARED((ns, LD), F32),
        pltpu.VMEM_SHARED((ns,), F32),
        pltpu.VMEM_SHARED((ns,), F32),
        pltpu.VMEM_SHARED((ns,), F32),
        pltpu.VMEM((ECHUNK, LD), F32),
        pltpu.VMEM((ECHUNK,), I32),
        pltpu.VMEM((ECHUNK,), I32),
        pltpu.VMEM((ECHUNK,), I32),
        pltpu.VMEM((ECHUNK,), I32),
        pltpu.VMEM((ECHUNK,), F32),
        pltpu.VMEM((ECHUNK,), F32),
        pltpu.VMEM((ECHUNK,), F32),
        pltpu.VMEM((ECHUNK,), F32),
        pltpu.VMEM((ECHUNK,), F32),
        pltpu.VMEM((ECHUNK,), F32),
        pltpu.VMEM((ECHUNK,), F32),
        pltpu.VMEM((ECHUNK,), F32),
        pltpu.VMEM((ECHUNK,), F32),
        pltpu.VMEM((16, LD), F32),
        pltpu.VMEM((ZF,), F32),
        pltpu.SemaphoreType.DMA,
    ]
    f = pl.kernel(
        functools.partial(_conv_body, ns, ept, with_pos, rows_by_dst),
        out_type=out_type,
        mesh=_sc_mesh(),
        scratch_types=scr,
    )
    px, py, pz = pxyz if with_pos else (w, w, w)
    return f(x, px, py, pz, w, d0, d1, a0, a1, src_pad, dst_pad)


GCH = 80


def _compact_body(ns_out, hp0_hbm, hp1_hbm, px0, px1, py0, py1, pz0, pz1,
                  aw0, aw1, mid_hbm, h_out, px_o, py_o, pz_o, w_o,
                  mid_b, r0_b, r1_b, s0_b, s1_b):
    wid = _wid()
    nrw = ns_out // NW
    base = wid * nrw
    for j0 in range(0, nrw, GCH):
        off = base + j0
        pltpu.sync_copy(mid_hbm.at[pl.ds(off, GCH)], mid_b)
        pltpu.sync_copy(hp0_hbm.at[mid_b], r0_b)
        pltpu.sync_copy(hp1_hbm.at[mid_b], r1_b)

        def add_rows(i, _):
            for k in range(8):
                cs = pl.ds(16 * k, 16)
                r0_b[i, cs] = r0_b[i, cs] + r1_b[i, cs]
            return ()
        lax.fori_loop(0, GCH, add_rows, ())
        pltpu.sync_copy(r0_b, h_out.at[pl.ds(off, GCH)])

        for (a_, b_, o_, eps) in ((px0, px1, px_o, 0.0),
                                  (py0, py1, py_o, 0.0),
                                  (pz0, pz1, pz_o, 0.0),
                                  (aw0, aw1, w_o, 1e-12)):
            pltpu.sync_copy(a_.at[mid_b], s0_b)
            pltpu.sync_copy(b_.at[mid_b], s1_b)

            def add_s(i, _):
                cs = pl.ds(16 * i, 16)
                s0_b[cs] = s0_b[cs] + s1_b[cs] + eps
                return ()
            lax.fori_loop(0, GCH // 16, add_s, ())
            pltpu.sync_copy(s0_b, o_.at[pl.ds(off, GCH)])


def _sc_compact(hp, pxp, pyp, pzp, awp, mid_pad):
    ns_out = mid_pad.shape[0]
    f = pl.kernel(
        functools.partial(_compact_body, ns_out),
        out_type=[
            jax.ShapeDtypeStruct((ns_out, LD), F32),
            jax.ShapeDtypeStruct((ns_out,), F32),
            jax.ShapeDtypeStruct((ns_out,), F32),
            jax.ShapeDtypeStruct((ns_out,), F32),
            jax.ShapeDtypeStruct((ns_out,), F32),
        ],
        mesh=_sc_mesh(),
        scratch_types=[
            pltpu.VMEM((GCH,), I32),
            pltpu.VMEM((GCH, LD), F32),
            pltpu.VMEM((GCH, LD), F32),
            pltpu.VMEM((GCH,), F32),
            pltpu.VMEM((GCH,), F32),
        ],
    )
    return f(hp[0], hp[1], pxp[0], pxp[1], pyp[0], pyp[1], pzp[0], pzp[1],
             awp[0], awp[1], mid_pad)


def _unpool_body(ns_out, hse_hbm, sel_hbm, u_out, sel_b, r_b):
    wid = _wid()
    nrw = ns_out // NW
    base = wid * nrw
    for j0 in range(0, nrw, GCH):
        off = base + j0
        pltpu.sync_copy(sel_hbm.at[pl.ds(off, GCH)], sel_b)
        pltpu.sync_copy(hse_hbm.at[sel_b], r_b)
        pltpu.sync_copy(r_b, u_out.at[pl.ds(off, GCH)])


def _sc_unpool(h_small_ext, sel_pad):
    ns_out = sel_pad.shape[0]
    f = pl.kernel(
        functools.partial(_unpool_body, ns_out),
        out_type=jax.ShapeDtypeStruct((ns_out, LD), F32),
        mesh=_sc_mesh(),
        scratch_types=[
            pltpu.VMEM((GCH,), I32),
            pltpu.VMEM((GCH, LD), F32),
        ],
    )
    return f(h_small_ext, sel_pad)


def _pre_body(h_ref, pos_ref, wa_ref, wb_ref, wp_ref, b1_ref, u_ref, v_ref):
    h = h_ref[...]
    posp = pos_ref[...] @ wp_ref[...]
    u_ref[...] = h @ wa_ref[...] + posp + b1_ref[...]
    v_ref[...] = h @ wb_ref[...] - posp


def _tc_pre(h, posr, wa, wb, wp, b1, blk=512):
    n = h.shape[0]
    return pl.pallas_call(
        _pre_body,
        grid=(_ceil(n, blk),),
        in_specs=[
            pl.BlockSpec((blk, LD), lambda i: (i, 0)),
            pl.BlockSpec((blk, 16), lambda i: (i, 0)),
            pl.BlockSpec((LD, LD), lambda i: (0, 0)),
            pl.BlockSpec((LD, LD), lambda i: (0, 0)),
            pl.BlockSpec((16, LD), lambda i: (0, 0)),
            pl.BlockSpec((1, LD), lambda i: (0, 0)),
        ],
        out_specs=[
            pl.BlockSpec((blk, LD), lambda i: (i, 0)),
            pl.BlockSpec((blk, LD), lambda i: (i, 0)),
        ],
        out_shape=[
            jax.ShapeDtypeStruct((n, LD), F32),
            jax.ShapeDtypeStruct((n, LD), F32),
        ],
    )(h, posr, wa, wb, wp, b1)


def _node_body(h_ref, s0_ref, s1_ref, d0_ref, d1_ref, w2e_ref, b2e_ref,
               wna_ref, wnb_ref, bn1_ref, wn2_ref, bn2_ref, res_ref, o_ref):
    h = h_ref[...]
    deg = d0_ref[...] + d1_ref[...]
    aggr = (s0_ref[0] + s1_ref[0]) @ w2e_ref[...] + deg * b2e_ref[...]
    z = jnp.maximum(h @ wna_ref[...] + aggr @ wnb_ref[...] + bn1_ref[...],
                    0.0)
    o_ref[...] = h + z @ wn2_ref[...] + bn2_ref[...] + res_ref[...]


def _tc_node(h, s_part, d0, d1, w2e, b2e, wna, wnb, bn1, wn2, bn2, res,
             blk=512):
    n = h.shape[0]
    return pl.pallas_call(
        _node_body,
        grid=(_ceil(n, blk),),
        in_specs=[
            pl.BlockSpec((blk, LD), lambda i: (i, 0)),
            pl.BlockSpec((1, blk, LD), lambda i: (0, i, 0)),
            pl.BlockSpec((1, blk, LD), lambda i: (1, i, 0)),
            pl.BlockSpec((blk, 1), lambda i: (i, 0)),
            pl.BlockSpec((blk, 1), lambda i: (i, 0)),
            pl.BlockSpec((LD, LD), lambda i: (0, 0)),
            pl.BlockSpec((1, LD), lambda i: (0, 0)),
            pl.BlockSpec((LD, LD), lambda i: (0, 0)),
            pl.BlockSpec((LD, LD), lambda i: (0, 0)),
            pl.BlockSpec((1, LD), lambda i: (0, 0)),
            pl.BlockSpec((LD, LD), lambda i: (0, 0)),
            pl.BlockSpec((1, LD), lambda i: (0, 0)),
            pl.BlockSpec((blk, LD), lambda i: (i, 0)),
        ],
        out_specs=pl.BlockSpec((blk, LD), lambda i: (i, 0)),
        out_shape=jax.ShapeDtypeStruct((n, LD), F32),
    )(h, s_part, s_part, d0[:, None], d1[:, None], w2e, b2e, wna, wnb, bn1,
      wn2, bn2, res)


def _add_body(a_ref, b_ref, o_ref):
    o_ref[...] = a_ref[0] + b_ref[0]


def _tc_add2(parts, blk=512):
    n = parts.shape[1]
    return pl.pallas_call(
        _add_body,
        grid=(_ceil(n, blk),),
        in_specs=[
            pl.BlockSpec((1, blk, LD), lambda i: (0, i, 0)),
            pl.BlockSpec((1, blk, LD), lambda i: (1, i, 0)),
        ],
        out_specs=pl.BlockSpec((blk, LD), lambda i: (i, 0)),
        out_shape=jax.ShapeDtypeStruct((n, LD), F32),
    )(parts, parts)


def _split_gmp_params(p):
    w1e = p["edge"]["W1"]
    wa = w1e[:LD]
    wb = w1e[LD:2 * LD]
    wp = jnp.pad(w1e[2 * LD:2 * LD + 3], ((0, 13), (0, 0)))
    wd = w1e[2 * LD + 3]
    b1 = p["edge"]["b1"][None, :]
    w2e = p["edge"]["W2"]
    b2e = p["edge"]["b2"][None, :]
    wn1 = p["node"]["W1"]
    return (wa, wb, wp, wd, b1, w2e, b2e, wn1[:LD], wn1[LD:],
            p["node"]["b1"][None, :], p["node"]["W2"],
            p["node"]["b2"][None, :])


def _pad_edges(src, dst, n):
    e = src.shape[0]
    epad = _ceil(e, NW * ECHUNK) * NW * ECHUNK
    fill = jnp.full((epad - e,), n, I32)
    return jnp.concatenate([src, fill]), jnp.concatenate([dst, fill])


def _gmp_sc(p, h_pad, posr, pxyz, srcp, dstp, res=None):
    (wa, wb, wp, wd, b1, w2e, b2e, wna, wnb, bn1, wn2, bn2) = \
        _split_gmp_params(p)
    u, v = _tc_pre(h_pad, posr, wa, wb, wp, b1)
    s_part, dd0, dd1 = _sc_edge(u, v, pxyz, srcp, dstp, wd)
    if res is None:
        res = jnp.zeros_like(h_pad)
    return _tc_node(h_pad, s_part, dd0, dd1, w2e, b2e, wna, wnb, bn1, wn2,
                    bn2, res)


def _pad_rows(x, ns):
    return jnp.pad(x, ((0, ns - x.shape[0]), (0, 0)))


def _pad_flat(x, ns):
    return jnp.pad(x, (0, ns - x.shape[0]))


def kernel(h, pos, params, m_gs_0, m_gs_1, m_gs_2, m_ids_0, m_ids_1):
    m_gs = [m_gs_0, m_gs_1, m_gs_2]
    m_ids = [m_ids_0, m_ids_1]
    nlist = [10000, 5000, 2500]

    ns0 = NS_TAB[10000]
    h_pad = _pad_rows(h, ns0)
    posr = jnp.pad(pos, ((0, ns0 - pos.shape[0]), (0, 13)))
    pxyz = (posr[:, 0], posr[:, 1], posr[:, 2])
    w = _pad_flat(jnp.ones((10000,), F32), ns0)

    down_hs, down_geo, down_sc, pads = [], [], [], []
    for i in range(2):
        n = nlist[i]
        ns = NS_TAB[n]
        nsn = NS_TAB[nlist[i + 1]]
        srcp, dstp = _pad_edges(m_gs[i][0], m_gs[i][1], n)
        pads.append((srcp, dstp))
        h_pad = _gmp_sc(params["down"][i], h_pad, posr, pxyz, srcp, dstp)
        down_hs.append(h_pad)
        down_geo.append((posr, pxyz))
        d0, d1 = _sc_prep1(srcp, ns)
        a0, a1 = _sc_prep2(w, d0, d1, srcp, dstp)
        down_sc.append((w, d0, d1, a0, a1))
        hp, pxf, pyf, pzf = _sc_conv(
            h_pad, pxyz, w, d0, d1, a0, a1, srcp, dstp, True, False)
        mid_pad = _pad_flat(m_ids[i], nsn)
        h_pad, px, py, pz, w = _sc_compact(
            hp, (pxf[:ns], pxf[ns:]), (pyf[:ns], pyf[ns:]),
            (pzf[:ns], pzf[ns:]), (a0, a1), mid_pad)
        posr = jnp.concatenate(
            [px[:, None], py[:, None], pz[:, None],
             jnp.zeros((nsn, 13), F32)], axis=1)
        pxyz = (px, py, pz)

    n2 = nlist[2]
    ns2 = NS_TAB[n2]
    srcp, dstp = _pad_edges(m_gs[2][0], m_gs[2][1], n2)
    h_pad = _gmp_sc(params["bottom"], h_pad, posr, pxyz, srcp, dstp)

    for i in range(2):
        li = 1 - i
        n = nlist[li]
        ns = NS_TAB[n]
        ns_small = NS_TAB[nlist[li + 1]]
        srcp, dstp = pads[li]
        posr, pxyz = down_geo[li]
        w_l, d0, d1, a0, a1 = down_sc[li]
        hse = jnp.concatenate([h_pad, jnp.zeros((LD, LD), F32)], axis=0)
        zrow = ns_small
        sel = jnp.full((ns,), zrow, I32).at[m_ids[li]].set(
            jnp.arange(nlist[li + 1], dtype=I32))
        u_arr = _sc_unpool(hse, sel)
        hu = _sc_conv(u_arr, None, w_l, d0, d1, a0, a1, srcp, dstp,
                      False, True)[0]
        h_uc = _tc_add2(hu)
        h_pad = _gmp_sc(params["up"][i], h_uc, posr, pxyz, srcp, dstp,
                        res=down_hs[li])
    return h_pad[:10000]


# speedup vs baseline: 3.4150x; 1.0742x over previous
import functools
import jax
import jax.numpy as jnp
from jax import lax
from jax.experimental import pallas as pl
from jax.experimental.pallas import tpu as pltpu
from jax.experimental.pallas import tpu_sc as plsc

LD = 128
NCORE = 2
NSUB = 16
NW = NCORE * NSUB
ECHUNK = 128
F32 = jnp.float32
I32 = jnp.int32

NS_TAB = {10000: 10240, 5000: 5120, 2500: 2560}


def _ceil(a, b):
    return (a + b - 1) // b


def _sc_mesh():
    return plsc.VectorSubcoreMesh(core_axis_name="c", subcore_axis_name="s",
                                  num_cores=NCORE, num_subcores=NSUB)


def _wid():
    return lax.axis_index("s") * NCORE + lax.axis_index("c")


def _fill_rows(buf_ref, val):
    def zrow(i, _):
        for k in range(buf_ref.shape[1] // 16):
            buf_ref[i, pl.ds(16 * k, 16)] = jnp.full((16,), val, F32)
        return ()
    lax.fori_loop(0, buf_ref.shape[0], zrow, ())


def _fill_flat(buf_ref, val):
    def zi(i, _):
        buf_ref[pl.ds(16 * i, 16)] = jnp.full((16,), val, F32)
        return ()
    lax.fori_loop(0, buf_ref.shape[0] // 16, zi, ())


ZF = 512


def _zero_flat_sh(sh_ref, zflat_ref):
    n = sh_ref.shape[0]
    def cp(j, _):
        pltpu.sync_copy(zflat_ref, sh_ref.at[pl.ds(j * ZF, ZF)])
        return ()
    lax.fori_loop(0, n // ZF, cp, ())


def _zero_rows(sh_ref, row0, nrows, zbuf_ref):
    zr = zbuf_ref.shape[0]
    def cp(j, _):
        pltpu.sync_copy(zbuf_ref, sh_ref.at[pl.ds(row0 + j * zr, zr)])
        return ()
    lax.fori_loop(0, nrows // zr, cp, ())


def _rsqrt_nr(d2v):
    i0 = lax.bitcast_convert_type(d2v, I32)
    y = lax.bitcast_convert_type(0x5F3759DF - (i0 >> 1), F32)
    for _ in range(3):
        y = y * (1.5 - 0.5 * d2v * y * y)
    return y


def _bcast_lane(vec, e):
    return vec.at[jnp.full((16,), e, I32)].get(mode="promise_in_bounds")


def _edge_body(ns, ept, ec, u_hbm, v_hbm, px_hbm, py_hbm, pz_hbm, src_hbm,
               dst_hbm, wd_hbm, s_out, d_out, s_sh, d_sh,
               u_a, v_a, si_a, di_a, pxs_a, pys_a, pzs_a, pxd_a, pyd_a,
               pzd_a, u_c, v_c, si_c, di_c, pxs_c, pys_c, pzs_c, pxd_c,
               pyd_c, pzd_c, wd_b, ones_b, zbuf, zflat, gsem_a, gsem_c,
               ssem_a, ssem_c):
    cid = lax.axis_index("c")
    sid = lax.axis_index("s")
    wid = _wid()
    nr = ns // NSUB

    bufA = (u_a, v_a, si_a, di_a, pxs_a, pys_a, pzs_a, pxd_a, pyd_a, pzd_a)
    bufB = (u_c, v_c, si_c, di_c, pxs_c, pys_c, pzs_c, pxd_c, pyd_c, pzd_c)

    _fill_rows(zbuf, 0.0)
    _zero_rows(s_sh, sid * nr, nr, zbuf)
    _fill_flat(ones_b, 1.0)
    @pl.when(sid == 0)
    def _():
        _fill_flat(zflat, 0.0)
        _zero_flat_sh(d_sh, zflat)
    pltpu.sync_copy(wd_hbm, wd_b)
    plsc.subcore_barrier()

    wdv = [wd_b[pl.ds(16 * k, 16)] for k in range(8)]
    base = wid * ept
    nc = ept // ec

    def stage_idx(jc, buf):
        off = base + jc * ec
        d1_ = pltpu.async_copy(src_hbm.at[pl.ds(off, ec)], buf[2], gsem_a)
        d2_ = pltpu.async_copy(dst_hbm.at[pl.ds(off, ec)], buf[3], gsem_a)
        d1_.wait()
        d2_.wait()

    def gather_args(buf):
        u_b, v_b, si_b, di_b, pxs, pys, pzs, pxd, pyd, pzd = buf
        return [(u_hbm.at[si_b], u_b), (v_hbm.at[di_b], v_b),
                (px_hbm.at[si_b], pxs), (py_hbm.at[si_b], pys),
                (pz_hbm.at[si_b], pzs), (px_hbm.at[di_b], pxd),
                (py_hbm.at[di_b], pyd), (pz_hbm.at[di_b], pzd)]

    def fire_gathers(buf, sem):
        for s_, d_ in gather_args(buf):
            pltpu.async_copy(s_, d_, sem)

    def drain_gathers(buf, sem):
        for s_, d_ in gather_args(buf):
            pltpu.make_async_copy(s_, d_, sem).wait()

    def fire_scatters(buf, sem):
        pltpu.async_copy(buf[0], s_sh.at[buf[3]], sem, add=True)
        pltpu.async_copy(ones_b, d_sh.at[buf[3]], sem, add=True)

    def drain_scatters(buf, sem):
        pltpu.make_async_copy(buf[0], s_sh.at[buf[3]], sem).wait()
        pltpu.make_async_copy(ones_b, d_sh.at[buf[3]], sem).wait()

    def compute(buf):
        u_b, v_b, si_b, di_b, pxs, pys, pzs, pxd, pyd, pzd = buf

        def group(g, _):
            gb = g * 16
            cs16 = pl.ds(gb, 16)
            dx = pxs[cs16] - pxd[cs16]
            dy = pys[cs16] - pyd[cs16]
            dz = pzs[cs16] - pzd[cs16]
            d2v = dx * dx + dy * dy + dz * dz + 1e-12
            dist = d2v * _rsqrt_nr(d2v)
            for e in range(16):
                row = gb + e
                db = _bcast_lane(dist, e)
                for k in range(8):
                    cs = pl.ds(16 * k, 16)
                    pre = u_b[row, cs] + v_b[row, cs] + db * wdv[k]
                    u_b[row, cs] = jnp.maximum(pre, 0.0)
            return ()
        lax.fori_loop(0, ec // 16, group, ())

    stage_idx(0, bufA)
    fire_gathers(bufA, gsem_a)

    def pair(t, _):
        j0 = 2 * t
        drain_gathers(bufA, gsem_a)
        @pl.when(t > 0)
        def _():
            drain_scatters(bufB, ssem_c)
        stage_idx(j0 + 1, bufB)
        fire_gathers(bufB, gsem_c)
        compute(bufA)
        fire_scatters(bufA, ssem_a)
        drain_gathers(bufB, gsem_c)
        drain_scatters(bufA, ssem_a)
        @pl.when(j0 + 2 < nc)
        def _():
            stage_idx(j0 + 2, bufA)
            fire_gathers(bufA, gsem_a)
        compute(bufB)
        fire_scatters(bufB, ssem_c)
        return ()
    lax.fori_loop(0, nc // 2, pair, ())
    drain_scatters(bufB, ssem_c)
    plsc.subcore_barrier()

    r0 = sid * nr
    pltpu.sync_copy(s_sh.at[pl.ds(r0, nr)], s_out.at[cid, pl.ds(r0, nr)])
    @pl.when(sid == 0)
    def _():
        pltpu.sync_copy(d_sh, d_out.at[pl.ds(cid * ns, ns)])


def _sc_edge(u, v, pxyz, src_pad, dst_pad, wd, ec=64):
    ns = u.shape[0]
    ept = src_pad.shape[0] // NW
    f = pl.kernel(
        functools.partial(_edge_body, ns, ept, ec),
        out_type=[
            jax.ShapeDtypeStruct((NCORE, ns, LD), F32),
            jax.ShapeDtypeStruct((NCORE * ns,), F32),
        ],
        mesh=_sc_mesh(),
        scratch_types=(
            [pltpu.VMEM_SHARED((ns, LD), F32),
             pltpu.VMEM_SHARED((ns,), F32)] +
            ([pltpu.VMEM((ec, LD), F32), pltpu.VMEM((ec, LD), F32),
              pltpu.VMEM((ec,), I32), pltpu.VMEM((ec,), I32)]
             + [pltpu.VMEM((ec,), F32)] * 6) * 2
            + [pltpu.VMEM((LD,), F32), pltpu.VMEM((ec,), F32),
               pltpu.VMEM((16, LD), F32), pltpu.VMEM((ZF,), F32),
               pltpu.SemaphoreType.DMA, pltpu.SemaphoreType.DMA,
               pltpu.SemaphoreType.DMA, pltpu.SemaphoreType.DMA]
        ),
    )
    s_part, d_flat = f(u, v, pxyz[0], pxyz[1], pxyz[2], src_pad, dst_pad, wd)
    return s_part, d_flat[:ns], d_flat[ns:]


def _prep1_body(ns, ept, src_hbm, d_out, d_sh, si_b, ones_b, zflat):
    cid = lax.axis_index("c")
    sid = lax.axis_index("s")
    wid = _wid()
    _fill_flat(ones_b, 1.0)
    @pl.when(sid == 0)
    def _():
        _fill_flat(zflat, 0.0)
        _zero_flat_sh(d_sh, zflat)
    plsc.subcore_barrier()
    base = wid * ept

    def chunk(jc, _):
        pltpu.sync_copy(src_hbm.at[pl.ds(base + jc * ECHUNK, ECHUNK)], si_b)
        pltpu.sync_copy(ones_b, d_sh.at[si_b], add=True)
        return ()
    lax.fori_loop(0, ept // ECHUNK, chunk, ())
    plsc.subcore_barrier()
    @pl.when(sid == 0)
    def _():
        pltpu.sync_copy(d_sh, d_out.at[pl.ds(cid * ns, ns)])


def _sc_prep1(src_pad, ns):
    ept = src_pad.shape[0] // NW
    f = pl.kernel(
        functools.partial(_prep1_body, ns, ept),
        out_type=jax.ShapeDtypeStruct((NCORE * ns,), F32),
        mesh=_sc_mesh(),
        scratch_types=[
            pltpu.VMEM_SHARED((ns,), F32),
            pltpu.VMEM((ECHUNK,), I32),
            pltpu.VMEM((ECHUNK,), F32),
            pltpu.VMEM((ZF,), F32),
        ],
    )
    d_flat = f(src_pad)
    return d_flat[:ns], d_flat[ns:]


def _prep2_body(ns, ept, w_hbm, d0_hbm, d1_hbm, src_hbm, dst_hbm,
                a_out, a_sh, si_b, di_b, wg, g0, g1, wts_b, zflat, sem):
    cid = lax.axis_index("c")
    sid = lax.axis_index("s")
    wid = _wid()
    @pl.when(sid == 0)
    def _():
        _fill_flat(zflat, 0.0)
        _zero_flat_sh(a_sh, zflat)
    plsc.subcore_barrier()
    base = wid * ept

    def chunk(jc, _):
        off = base + jc * ECHUNK
        d1_ = pltpu.async_copy(src_hbm.at[pl.ds(off, ECHUNK)], si_b, sem)
        d2_ = pltpu.async_copy(dst_hbm.at[pl.ds(off, ECHUNK)], di_b, sem)
        d1_.wait()
        d2_.wait()
        descs = [
            pltpu.async_copy(w_hbm.at[si_b], wg, sem),
            pltpu.async_copy(d0_hbm.at[si_b], g0, sem),
            pltpu.async_copy(d1_hbm.at[si_b], g1, sem),
        ]
        for d_ in descs:
            d_.wait()

        def group(g, _):
            cs = pl.ds(g * 16, 16)
            deg = jnp.maximum(g0[cs] + g1[cs], 1.0)
            wts_b[cs] = wg[cs] / deg
            return ()
        lax.fori_loop(0, ECHUNK // 16, group, ())
        pltpu.sync_copy(wts_b, a_sh.at[di_b], add=True)
        return ()
    lax.fori_loop(0, ept // ECHUNK, chunk, ())
    plsc.subcore_barrier()
    @pl.when(sid == 0)
    def _():
        pltpu.sync_copy(a_sh, a_out.at[pl.ds(cid * ns, ns)])


def _sc_prep2(w, d0, d1, src_pad, dst_pad):
    ns = w.shape[0]
    ept = src_pad.shape[0] // NW
    f = pl.kernel(
        functools.partial(_prep2_body, ns, ept),
        out_type=jax.ShapeDtypeStruct((NCORE * ns,), F32),
        mesh=_sc_mesh(),
        scratch_types=[
            pltpu.VMEM_SHARED((ns,), F32),
            pltpu.VMEM((ECHUNK,), I32),
            pltpu.VMEM((ECHUNK,), I32),
            pltpu.VMEM((ECHUNK,), F32),
            pltpu.VMEM((ECHUNK,), F32),
            pltpu.VMEM((ECHUNK,), F32),
            pltpu.VMEM((ECHUNK,), F32),
            pltpu.VMEM((ZF,), F32),
            pltpu.SemaphoreType.DMA,
        ],
    )
    a_flat = f(w, d0, d1, src_pad, dst_pad)
    return a_flat[:ns], a_flat[ns:]


def _conv_body(ns, ept, with_pos, rows_by_dst, x_hbm, px_hbm, py_hbm, pz_hbm,
               w_hbm, d0_hbm, d1_hbm, a0_hbm, a1_hbm, src_hbm, dst_hbm,
               h_out, px_o, py_o, pz_o,
               h_sh, px_sh, py_sh, pz_sh, x_b, gi_b, sc_b, si_b, di_b,
               wg, g0, g1, a0g, a1g, ec_b, pxg, pyg, pzg, zbuf, zflat, sem):
    cid = lax.axis_index("c")
    sid = lax.axis_index("s")
    wid = _wid()
    nr = ns // NSUB

    _fill_rows(zbuf, 0.0)
    _zero_rows(h_sh, sid * nr, nr, zbuf)
    @pl.when(sid == 0)
    def _():
        _fill_flat(zflat, 0.0)
        if with_pos:
            _zero_flat_sh(px_sh, zflat)
            _zero_flat_sh(py_sh, zflat)
            _zero_flat_sh(pz_sh, zflat)
    plsc.subcore_barrier()
    base = wid * ept

    def chunk(jc, _):
        off = base + jc * ECHUNK
        d1_ = pltpu.async_copy(src_hbm.at[pl.ds(off, ECHUNK)], si_b, sem)
        d2_ = pltpu.async_copy(dst_hbm.at[pl.ds(off, ECHUNK)], di_b, sem)
        d1_.wait()
        d2_.wait()
        gi = di_b if rows_by_dst else si_b
        sc = si_b if rows_by_dst else di_b
        descs = [
            pltpu.async_copy(x_hbm.at[gi], x_b, sem),
            pltpu.async_copy(w_hbm.at[si_b], wg, sem),
            pltpu.async_copy(d0_hbm.at[si_b], g0, sem),
            pltpu.async_copy(d1_hbm.at[si_b], g1, sem),
            pltpu.async_copy(a0_hbm.at[di_b], a0g, sem),
            pltpu.async_copy(a1_hbm.at[di_b], a1g, sem),
        ]
        if with_pos:
            descs += [
                pltpu.async_copy(px_hbm.at[si_b], pxg, sem),
                pltpu.async_copy(py_hbm.at[si_b], pyg, sem),
                pltpu.async_copy(pz_hbm.at[si_b], pzg, sem),
            ]
        for d_ in descs:
            d_.wait()

        def group(g, _):
            gb = g * 16
            cs16 = pl.ds(gb, 16)
            deg = jnp.maximum(g0[cs16] + g1[cs16], 1.0)
            ecv = (wg[cs16] / deg) / (a0g[cs16] + a1g[cs16] + 1e-12)
            ec_b[cs16] = ecv
            if with_pos:
                pxg[cs16] = ecv * pxg[cs16]
                pyg[cs16] = ecv * pyg[cs16]
                pzg[cs16] = ecv * pzg[cs16]
            for e in range(16):
                row = gb + e
                eb = _bcast_lane(ecv, e)
                for k in range(8):
                    cs = pl.ds(16 * k, 16)
                    x_b[row, cs] = eb * x_b[row, cs]
            return ()
        lax.fori_loop(0, ECHUNK // 16, group, ())
        descs2 = [pltpu.async_copy(x_b, h_sh.at[sc], sem, add=True)]
        if with_pos:
            descs2 += [
                pltpu.async_copy(pxg, px_sh.at[sc], sem, add=True),
                pltpu.async_copy(pyg, py_sh.at[sc], sem, add=True),
                pltpu.async_copy(pzg, pz_sh.at[sc], sem, add=True),
            ]
        for d_ in descs2:
            d_.wait()
        return ()
    lax.fori_loop(0, ept // ECHUNK, chunk, ())
    plsc.subcore_barrier()

    r0 = sid * nr
    pltpu.sync_copy(h_sh.at[pl.ds(r0, nr)], h_out.at[cid, pl.ds(r0, nr)])
    if with_pos:
        @pl.when(sid == 0)
        def _():
            pltpu.sync_copy(px_sh, px_o.at[pl.ds(cid * ns, ns)])
            pltpu.sync_copy(py_sh, py_o.at[pl.ds(cid * ns, ns)])
            pltpu.sync_copy(pz_sh, pz_o.at[pl.ds(cid * ns, ns)])


def _sc_conv(x, pxyz, w, d0, d1, a0, a1, src_pad, dst_pad, with_pos,
             rows_by_dst):
    ns = x.shape[0]
    ept = src_pad.shape[0] // NW
    out_type = [jax.ShapeDtypeStruct((NCORE, ns, LD), F32)]
    out_type += [jax.ShapeDtypeStruct((NCORE * ns,), F32)] * 3
    scr = [
        pltpu.VMEM_SHARED((ns, LD), F32),
        pltpu.VMEM_SHARED((ns,), F32),
        pltpu.VMEM_SHARED((ns,), F32),
        pltpu.VMEM_SHARED((ns,), F32),
        pltpu.VMEM((ECHUNK, LD), F32),
        pltpu.VMEM((ECHUNK,), I32),
        pltpu.VMEM((ECHUNK,), I32),
        pltpu.VMEM((ECHUNK,), I32),
        pltpu.VMEM((ECHUNK,), I32),
        pltpu.VMEM((ECHUNK,), F32),
        pltpu.VMEM((ECHUNK,), F32),
        pltpu.VMEM((ECHUNK,), F32),
        pltpu.VMEM((ECHUNK,), F32),
        pltpu.VMEM((ECHUNK,), F32),
        pltpu.VMEM((ECHUNK,), F32),
        pltpu.VMEM((ECHUNK,), F32),
        pltpu.VMEM((ECHUNK,), F32),
        pltpu.VMEM((ECHUNK,), F32),
        pltpu.VMEM((16, LD), F32),
        pltpu.VMEM((ZF,), F32),
        pltpu.SemaphoreType.DMA,
    ]
    f = pl.kernel(
        functools.partial(_conv_body, ns, ept, with_pos, rows_by_dst),
        out_type=out_type,
        mesh=_sc_mesh(),
        scratch_types=scr,
    )
    px, py, pz = pxyz if with_pos else (w, w, w)
    return f(x, px, py, pz, w, d0, d1, a0, a1, src_pad, dst_pad)


GCH = 80


def _compact_body(ns_out, hp0_hbm, hp1_hbm, px0, px1, py0, py1, pz0, pz1,
                  aw0, aw1, mid_hbm, h_out, px_o, py_o, pz_o, w_o,
                  mid_b, r0_b, r1_b, s0_b, s1_b):
    wid = _wid()
    nrw = ns_out // NW
    base = wid * nrw
    for j0 in range(0, nrw, GCH):
        off = base + j0
        pltpu.sync_copy(mid_hbm.at[pl.ds(off, GCH)], mid_b)
        pltpu.sync_copy(hp0_hbm.at[mid_b], r0_b)
        pltpu.sync_copy(hp1_hbm.at[mid_b], r1_b)

        def add_rows(i, _):
            for k in range(8):
                cs = pl.ds(16 * k, 16)
                r0_b[i, cs] = r0_b[i, cs] + r1_b[i, cs]
            return ()
        lax.fori_loop(0, GCH, add_rows, ())
        pltpu.sync_copy(r0_b, h_out.at[pl.ds(off, GCH)])

        for (a_, b_, o_, eps) in ((px0, px1, px_o, 0.0),
                                  (py0, py1, py_o, 0.0),
                                  (pz0, pz1, pz_o, 0.0),
                                  (aw0, aw1, w_o, 1e-12)):
            pltpu.sync_copy(a_.at[mid_b], s0_b)
            pltpu.sync_copy(b_.at[mid_b], s1_b)

            def add_s(i, _):
                cs = pl.ds(16 * i, 16)
                s0_b[cs] = s0_b[cs] + s1_b[cs] + eps
                return ()
            lax.fori_loop(0, GCH // 16, add_s, ())
            pltpu.sync_copy(s0_b, o_.at[pl.ds(off, GCH)])


def _sc_compact(hp, pxp, pyp, pzp, awp, mid_pad):
    ns_out = mid_pad.shape[0]
    f = pl.kernel(
        functools.partial(_compact_body, ns_out),
        out_type=[
            jax.ShapeDtypeStruct((ns_out, LD), F32),
            jax.ShapeDtypeStruct((ns_out,), F32),
            jax.ShapeDtypeStruct((ns_out,), F32),
            jax.ShapeDtypeStruct((ns_out,), F32),
            jax.ShapeDtypeStruct((ns_out,), F32),
        ],
        mesh=_sc_mesh(),
        scratch_types=[
            pltpu.VMEM((GCH,), I32),
            pltpu.VMEM((GCH, LD), F32),
            pltpu.VMEM((GCH, LD), F32),
            pltpu.VMEM((GCH,), F32),
            pltpu.VMEM((GCH,), F32),
        ],
    )
    return f(hp[0], hp[1], pxp[0], pxp[1], pyp[0], pyp[1], pzp[0], pzp[1],
             awp[0], awp[1], mid_pad)


def _unpool_body(ns_out, hse_hbm, sel_hbm, u_out, sel_b, r_b):
    wid = _wid()
    nrw = ns_out // NW
    base = wid * nrw
    for j0 in range(0, nrw, GCH):
        off = base + j0
        pltpu.sync_copy(sel_hbm.at[pl.ds(off, GCH)], sel_b)
        pltpu.sync_copy(hse_hbm.at[sel_b], r_b)
        pltpu.sync_copy(r_b, u_out.at[pl.ds(off, GCH)])


def _sc_unpool(h_small_ext, sel_pad):
    ns_out = sel_pad.shape[0]
    f = pl.kernel(
        functools.partial(_unpool_body, ns_out),
        out_type=jax.ShapeDtypeStruct((ns_out, LD), F32),
        mesh=_sc_mesh(),
        scratch_types=[
            pltpu.VMEM((GCH,), I32),
            pltpu.VMEM((GCH, LD), F32),
        ],
    )
    return f(h_small_ext, sel_pad)


def _pre_body(h_ref, pos_ref, wa_ref, wb_ref, wp_ref, b1_ref, u_ref, v_ref):
    h = h_ref[...]
    posp = pos_ref[...] @ wp_ref[...]
    u_ref[...] = h @ wa_ref[...] + posp + b1_ref[...]
    v_ref[...] = h @ wb_ref[...] - posp


def _tc_pre(h, posr, wa, wb, wp, b1, blk=512):
    n = h.shape[0]
    return pl.pallas_call(
        _pre_body,
        grid=(_ceil(n, blk),),
        in_specs=[
            pl.BlockSpec((blk, LD), lambda i: (i, 0)),
            pl.BlockSpec((blk, 16), lambda i: (i, 0)),
            pl.BlockSpec((LD, LD), lambda i: (0, 0)),
            pl.BlockSpec((LD, LD), lambda i: (0, 0)),
            pl.BlockSpec((16, LD), lambda i: (0, 0)),
            pl.BlockSpec((1, LD), lambda i: (0, 0)),
        ],
        out_specs=[
            pl.BlockSpec((blk, LD), lambda i: (i, 0)),
            pl.BlockSpec((blk, LD), lambda i: (i, 0)),
        ],
        out_shape=[
            jax.ShapeDtypeStruct((n, LD), F32),
            jax.ShapeDtypeStruct((n, LD), F32),
        ],
    )(h, posr, wa, wb, wp, b1)


def _node_body(h_ref, s0_ref, s1_ref, d0_ref, d1_ref, w2e_ref, b2e_ref,
               wna_ref, wnb_ref, bn1_ref, wn2_ref, bn2_ref, res_ref, o_ref):
    h = h_ref[...]
    deg = d0_ref[...] + d1_ref[...]
    aggr = (s0_ref[0] + s1_ref[0]) @ w2e_ref[...] + deg * b2e_ref[...]
    z = jnp.maximum(h @ wna_ref[...] + aggr @ wnb_ref[...] + bn1_ref[...],
                    0.0)
    o_ref[...] = h + z @ wn2_ref[...] + bn2_ref[...] + res_ref[...]


def _tc_node(h, s_part, d0, d1, w2e, b2e, wna, wnb, bn1, wn2, bn2, res,
             blk=512):
    n = h.shape[0]
    return pl.pallas_call(
        _node_body,
        grid=(_ceil(n, blk),),
        in_specs=[
            pl.BlockSpec((blk, LD), lambda i: (i, 0)),
            pl.BlockSpec((1, blk, LD), lambda i: (0, i, 0)),
            pl.BlockSpec((1, blk, LD), lambda i: (1, i, 0)),
            pl.BlockSpec((blk, 1), lambda i: (i, 0)),
            pl.BlockSpec((blk, 1), lambda i: (i, 0)),
            pl.BlockSpec((LD, LD), lambda i: (0, 0)),
            pl.BlockSpec((1, LD), lambda i: (0, 0)),
            pl.BlockSpec((LD, LD), lambda i: (0, 0)),
            pl.BlockSpec((LD, LD), lambda i: (0, 0)),
            pl.BlockSpec((1, LD), lambda i: (0, 0)),
            pl.BlockSpec((LD, LD), lambda i: (0, 0)),
            pl.BlockSpec((1, LD), lambda i: (0, 0)),
            pl.BlockSpec((blk, LD), lambda i: (i, 0)),
        ],
        out_specs=pl.BlockSpec((blk, LD), lambda i: (i, 0)),
        out_shape=jax.ShapeDtypeStruct((n, LD), F32),
    )(h, s_part, s_part, d0[:, None], d1[:, None], w2e, b2e, wna, wnb, bn1,
      wn2, bn2, res)


def _add_body(a_ref, b_ref, o_ref):
    o_ref[...] = a_ref[0] + b_ref[0]


def _tc_add2(parts, blk=512):
    n = parts.shape[1]
    return pl.pallas_call(
        _add_body,
        grid=(_ceil(n, blk),),
        in_specs=[
            pl.BlockSpec((1, blk, LD), lambda i: (0, i, 0)),
            pl.BlockSpec((1, blk, LD), lambda i: (1, i, 0)),
        ],
        out_specs=pl.BlockSpec((blk, LD), lambda i: (i, 0)),
        out_shape=jax.ShapeDtypeStruct((n, LD), F32),
    )(parts, parts)


def _split_gmp_params(p):
    w1e = p["edge"]["W1"]
    wa = w1e[:LD]
    wb = w1e[LD:2 * LD]
    wp = jnp.pad(w1e[2 * LD:2 * LD + 3], ((0, 13), (0, 0)))
    wd = w1e[2 * LD + 3]
    b1 = p["edge"]["b1"][None, :]
    w2e = p["edge"]["W2"]
    b2e = p["edge"]["b2"][None, :]
    wn1 = p["node"]["W1"]
    return (wa, wb, wp, wd, b1, w2e, b2e, wn1[:LD], wn1[LD:],
            p["node"]["b1"][None, :], p["node"]["W2"],
            p["node"]["b2"][None, :])


def _pad_edges(src, dst, n):
    e = src.shape[0]
    g = NW * ECHUNK * 2
    epad = _ceil(e, g) * g
    fill = jnp.full((epad - e,), n, I32)
    return jnp.concatenate([src, fill]), jnp.concatenate([dst, fill])


def _gmp_sc(p, h_pad, posr, pxyz, srcp, dstp, res=None):
    (wa, wb, wp, wd, b1, w2e, b2e, wna, wnb, bn1, wn2, bn2) = \
        _split_gmp_params(p)
    u, v = _tc_pre(h_pad, posr, wa, wb, wp, b1)
    s_part, dd0, dd1 = _sc_edge(u, v, pxyz, srcp, dstp, wd)
    if res is None:
        res = jnp.zeros_like(h_pad)
    return _tc_node(h_pad, s_part, dd0, dd1, w2e, b2e, wna, wnb, bn1, wn2,
                    bn2, res)


def _pad_rows(x, ns):
    return jnp.pad(x, ((0, ns - x.shape[0]), (0, 0)))


def _pad_flat(x, ns):
    return jnp.pad(x, (0, ns - x.shape[0]))


def kernel(h, pos, params, m_gs_0, m_gs_1, m_gs_2, m_ids_0, m_ids_1):
    m_gs = [m_gs_0, m_gs_1, m_gs_2]
    m_ids = [m_ids_0, m_ids_1]
    nlist = [10000, 5000, 2500]

    ns0 = NS_TAB[10000]
    h_pad = _pad_rows(h, ns0)
    posr = jnp.pad(pos, ((0, ns0 - pos.shape[0]), (0, 13)))
    pxyz = (posr[:, 0], posr[:, 1], posr[:, 2])
    w = _pad_flat(jnp.ones((10000,), F32), ns0)

    down_hs, down_geo, down_sc, pads = [], [], [], []
    for i in range(2):
        n = nlist[i]
        ns = NS_TAB[n]
        nsn = NS_TAB[nlist[i + 1]]
        srcp, dstp = _pad_edges(m_gs[i][0], m_gs[i][1], n)
        pads.append((srcp, dstp))
        h_pad = _gmp_sc(params["down"][i], h_pad, posr, pxyz, srcp, dstp)
        down_hs.append(h_pad)
        down_geo.append((posr, pxyz))
        d0, d1 = _sc_prep1(srcp, ns)
        a0, a1 = _sc_prep2(w, d0, d1, srcp, dstp)
        down_sc.append((w, d0, d1, a0, a1))
        hp, pxf, pyf, pzf = _sc_conv(
            h_pad, pxyz, w, d0, d1, a0, a1, srcp, dstp, True, False)
        mid_pad = _pad_flat(m_ids[i], nsn)
        h_pad, px, py, pz, w = _sc_compact(
            hp, (pxf[:ns], pxf[ns:]), (pyf[:ns], pyf[ns:]),
            (pzf[:ns], pzf[ns:]), (a0, a1), mid_pad)
        posr = jnp.concatenate(
            [px[:, None], py[:, None], pz[:, None],
             jnp.zeros((nsn, 13), F32)], axis=1)
        pxyz = (px, py, pz)

    n2 = nlist[2]
    ns2 = NS_TAB[n2]
    srcp, dstp = _pad_edges(m_gs[2][0], m_gs[2][1], n2)
    h_pad = _gmp_sc(params["bottom"], h_pad, posr, pxyz, srcp, dstp)

    for i in range(2):
        li = 1 - i
        n = nlist[li]
        ns = NS_TAB[n]
        ns_small = NS_TAB[nlist[li + 1]]
        srcp, dstp = pads[li]
        posr, pxyz = down_geo[li]
        w_l, d0, d1, a0, a1 = down_sc[li]
        hse = jnp.concatenate([h_pad, jnp.zeros((LD, LD), F32)], axis=0)
        zrow = ns_small
        sel = jnp.full((ns,), zrow, I32).at[m_ids[li]].set(
            jnp.arange(nlist[li + 1], dtype=I32))
        u_arr = _sc_unpool(hse, sel)
        hu = _sc_conv(u_arr, None, w_l, d0, d1, a0, a1, srcp, dstp,
                      False, True)[0]
        h_uc = _tc_add2(hu)
        h_pad = _gmp_sc(params["up"][i], h_uc, posr, pxyz, srcp, dstp,
                        res=down_hs[li])
    return h_pad[:10000]


# speedup vs baseline: 3.5000x; 1.0249x over previous
import functools
import jax
import jax.numpy as jnp
from jax import lax
from jax.experimental import pallas as pl
from jax.experimental.pallas import tpu as pltpu
from jax.experimental.pallas import tpu_sc as plsc

LD = 128
NCORE = 2
NSUB = 16
NW = NCORE * NSUB
ECHUNK = 128
F32 = jnp.float32
I32 = jnp.int32

NS_TAB = {10000: 10240, 5000: 5120, 2500: 2560}


def _ceil(a, b):
    return (a + b - 1) // b


def _sc_mesh():
    return plsc.VectorSubcoreMesh(core_axis_name="c", subcore_axis_name="s",
                                  num_cores=NCORE, num_subcores=NSUB)


def _wid():
    return lax.axis_index("s") * NCORE + lax.axis_index("c")


def _fill_rows(buf_ref, val):
    def zrow(i, _):
        for k in range(buf_ref.shape[1] // 16):
            buf_ref[i, pl.ds(16 * k, 16)] = jnp.full((16,), val, F32)
        return ()
    lax.fori_loop(0, buf_ref.shape[0], zrow, ())


def _fill_flat(buf_ref, val):
    def zi(i, _):
        buf_ref[pl.ds(16 * i, 16)] = jnp.full((16,), val, F32)
        return ()
    lax.fori_loop(0, buf_ref.shape[0] // 16, zi, ())


ZF = 512


def _zero_flat_sh(sh_ref, zflat_ref):
    n = sh_ref.shape[0]
    def cp(j, _):
        pltpu.sync_copy(zflat_ref, sh_ref.at[pl.ds(j * ZF, ZF)])
        return ()
    lax.fori_loop(0, n // ZF, cp, ())


def _zero_rows(sh_ref, row0, nrows, zbuf_ref):
    zr = zbuf_ref.shape[0]
    def cp(j, _):
        pltpu.sync_copy(zbuf_ref, sh_ref.at[pl.ds(row0 + j * zr, zr)])
        return ()
    lax.fori_loop(0, nrows // zr, cp, ())


def _rsqrt_nr(d2v):
    i0 = lax.bitcast_convert_type(d2v, I32)
    y = lax.bitcast_convert_type(0x5F3759DF - (i0 >> 1), F32)
    for _ in range(3):
        y = y * (1.5 - 0.5 * d2v * y * y)
    return y


def _bcast_lane(vec, e):
    return vec.at[jnp.full((16,), e, I32)].get(mode="promise_in_bounds")


def _edge_body(ns, ept, ec, u_hbm, v_hbm, px_hbm, py_hbm, pz_hbm, src_hbm,
               dst_hbm, wd_hbm, s_out, d_out, s_sh, d_sh,
               u_a, v_a, si_a, di_a, pxs_a, pys_a, pzs_a, pxd_a, pyd_a,
               pzd_a, u_c, v_c, si_c, di_c, pxs_c, pys_c, pzs_c, pxd_c,
               pyd_c, pzd_c, wd_b, ones_b, zbuf, zflat, gsem_a, gsem_c,
               ssem_a, ssem_c):
    cid = lax.axis_index("c")
    sid = lax.axis_index("s")
    wid = _wid()
    nr = ns // NSUB

    bufA = (u_a, v_a, si_a, di_a, pxs_a, pys_a, pzs_a, pxd_a, pyd_a, pzd_a)
    bufB = (u_c, v_c, si_c, di_c, pxs_c, pys_c, pzs_c, pxd_c, pyd_c, pzd_c)

    _fill_rows(zbuf, 0.0)
    _zero_rows(s_sh, sid * nr, nr, zbuf)
    _fill_flat(ones_b, 1.0)
    @pl.when(sid == 0)
    def _():
        _fill_flat(zflat, 0.0)
        _zero_flat_sh(d_sh, zflat)
    pltpu.sync_copy(wd_hbm, wd_b)
    plsc.subcore_barrier()

    wdv = [wd_b[pl.ds(16 * k, 16)] for k in range(8)]
    base = wid * ept
    nc = ept // ec

    def stage_idx(jc, buf):
        off = base + jc * ec
        d1_ = pltpu.async_copy(src_hbm.at[pl.ds(off, ec)], buf[2], gsem_a)
        d2_ = pltpu.async_copy(dst_hbm.at[pl.ds(off, ec)], buf[3], gsem_a)
        d1_.wait()
        d2_.wait()

    def gather_args(buf):
        u_b, v_b, si_b, di_b, pxs, pys, pzs, pxd, pyd, pzd = buf
        return [(u_hbm.at[si_b], u_b), (v_hbm.at[di_b], v_b),
                (px_hbm.at[si_b], pxs), (py_hbm.at[si_b], pys),
                (pz_hbm.at[si_b], pzs), (px_hbm.at[di_b], pxd),
                (py_hbm.at[di_b], pyd), (pz_hbm.at[di_b], pzd)]

    def fire_gathers(buf, sem):
        for s_, d_ in gather_args(buf):
            pltpu.async_copy(s_, d_, sem)

    def drain_gathers(buf, sem):
        for s_, d_ in gather_args(buf):
            pltpu.make_async_copy(s_, d_, sem).wait()

    def fire_scatters(buf, sem):
        pltpu.async_copy(buf[0], s_sh.at[buf[3]], sem, add=True)
        pltpu.async_copy(ones_b, d_sh.at[buf[3]], sem, add=True)

    def drain_scatters(buf, sem):
        pltpu.make_async_copy(buf[0], s_sh.at[buf[3]], sem).wait()
        pltpu.make_async_copy(ones_b, d_sh.at[buf[3]], sem).wait()

    def compute(buf):
        u_b, v_b, si_b, di_b, pxs, pys, pzs, pxd, pyd, pzd = buf

        def group(g, _):
            gb = g * 16
            cs16 = pl.ds(gb, 16)
            dx = pxs[cs16] - pxd[cs16]
            dy = pys[cs16] - pyd[cs16]
            dz = pzs[cs16] - pzd[cs16]
            d2v = dx * dx + dy * dy + dz * dz + 1e-12
            dist = d2v * _rsqrt_nr(d2v)
            for e in range(16):
                row = gb + e
                db = _bcast_lane(dist, e)
                for k in range(8):
                    cs = pl.ds(16 * k, 16)
                    pre = u_b[row, cs] + v_b[row, cs] + db * wdv[k]
                    u_b[row, cs] = jnp.maximum(pre, 0.0)
            return ()
        lax.fori_loop(0, ec // 16, group, ())

    stage_idx(0, bufA)
    fire_gathers(bufA, gsem_a)

    def pair(t, _):
        j0 = 2 * t
        drain_gathers(bufA, gsem_a)
        @pl.when(t > 0)
        def _():
            drain_scatters(bufB, ssem_c)
        stage_idx(j0 + 1, bufB)
        fire_gathers(bufB, gsem_c)
        compute(bufA)
        fire_scatters(bufA, ssem_a)
        drain_gathers(bufB, gsem_c)
        drain_scatters(bufA, ssem_a)
        @pl.when(j0 + 2 < nc)
        def _():
            stage_idx(j0 + 2, bufA)
            fire_gathers(bufA, gsem_a)
        compute(bufB)
        fire_scatters(bufB, ssem_c)
        return ()
    lax.fori_loop(0, nc // 2, pair, ())
    drain_scatters(bufB, ssem_c)
    plsc.subcore_barrier()

    r0 = sid * nr
    pltpu.sync_copy(s_sh.at[pl.ds(r0, nr)], s_out.at[cid, pl.ds(r0, nr)])
    @pl.when(sid == 0)
    def _():
        pltpu.sync_copy(d_sh, d_out.at[pl.ds(cid * ns, ns)])


def _sc_edge(u, v, pxyz, src_pad, dst_pad, wd, ec=64):
    ns = u.shape[0]
    ept = src_pad.shape[0] // NW
    f = pl.kernel(
        functools.partial(_edge_body, ns, ept, ec),
        out_type=[
            jax.ShapeDtypeStruct((NCORE, ns, LD), F32),
            jax.ShapeDtypeStruct((NCORE * ns,), F32),
        ],
        mesh=_sc_mesh(),
        scratch_types=(
            [pltpu.VMEM_SHARED((ns, LD), F32),
             pltpu.VMEM_SHARED((ns,), F32)] +
            ([pltpu.VMEM((ec, LD), F32), pltpu.VMEM((ec, LD), F32),
              pltpu.VMEM((ec,), I32), pltpu.VMEM((ec,), I32)]
             + [pltpu.VMEM((ec,), F32)] * 6) * 2
            + [pltpu.VMEM((LD,), F32), pltpu.VMEM((ec,), F32),
               pltpu.VMEM((16, LD), F32), pltpu.VMEM((ZF,), F32),
               pltpu.SemaphoreType.DMA, pltpu.SemaphoreType.DMA,
               pltpu.SemaphoreType.DMA, pltpu.SemaphoreType.DMA]
        ),
    )
    s_part, d_flat = f(u, v, pxyz[0], pxyz[1], pxyz[2], src_pad, dst_pad, wd)
    return s_part, d_flat[:ns], d_flat[ns:]


def _prep1_body(ns, ept, src_hbm, d_out, d_sh, si_b, ones_b, zflat):
    cid = lax.axis_index("c")
    sid = lax.axis_index("s")
    wid = _wid()
    _fill_flat(ones_b, 1.0)
    @pl.when(sid == 0)
    def _():
        _fill_flat(zflat, 0.0)
        _zero_flat_sh(d_sh, zflat)
    plsc.subcore_barrier()
    base = wid * ept

    def chunk(jc, _):
        pltpu.sync_copy(src_hbm.at[pl.ds(base + jc * ECHUNK, ECHUNK)], si_b)
        pltpu.sync_copy(ones_b, d_sh.at[si_b], add=True)
        return ()
    lax.fori_loop(0, ept // ECHUNK, chunk, ())
    plsc.subcore_barrier()
    @pl.when(sid == 0)
    def _():
        pltpu.sync_copy(d_sh, d_out.at[pl.ds(cid * ns, ns)])


def _sc_prep1(src_pad, ns):
    ept = src_pad.shape[0] // NW
    f = pl.kernel(
        functools.partial(_prep1_body, ns, ept),
        out_type=jax.ShapeDtypeStruct((NCORE * ns,), F32),
        mesh=_sc_mesh(),
        scratch_types=[
            pltpu.VMEM_SHARED((ns,), F32),
            pltpu.VMEM((ECHUNK,), I32),
            pltpu.VMEM((ECHUNK,), F32),
            pltpu.VMEM((ZF,), F32),
        ],
    )
    d_flat = f(src_pad)
    return d_flat[:ns], d_flat[ns:]


def _prep2_body(ns, ept, w_hbm, d0_hbm, d1_hbm, src_hbm, dst_hbm,
                a_out, a_sh, si_b, di_b, wg, g0, g1, wts_b, zflat, sem):
    cid = lax.axis_index("c")
    sid = lax.axis_index("s")
    wid = _wid()
    @pl.when(sid == 0)
    def _():
        _fill_flat(zflat, 0.0)
        _zero_flat_sh(a_sh, zflat)
    plsc.subcore_barrier()
    base = wid * ept

    def chunk(jc, _):
        off = base + jc * ECHUNK
        d1_ = pltpu.async_copy(src_hbm.at[pl.ds(off, ECHUNK)], si_b, sem)
        d2_ = pltpu.async_copy(dst_hbm.at[pl.ds(off, ECHUNK)], di_b, sem)
        d1_.wait()
        d2_.wait()
        descs = [
            pltpu.async_copy(w_hbm.at[si_b], wg, sem),
            pltpu.async_copy(d0_hbm.at[si_b], g0, sem),
            pltpu.async_copy(d1_hbm.at[si_b], g1, sem),
        ]
        for d_ in descs:
            d_.wait()

        def group(g, _):
            cs = pl.ds(g * 16, 16)
            deg = jnp.maximum(g0[cs] + g1[cs], 1.0)
            wts_b[cs] = wg[cs] / deg
            return ()
        lax.fori_loop(0, ECHUNK // 16, group, ())
        pltpu.sync_copy(wts_b, a_sh.at[di_b], add=True)
        return ()
    lax.fori_loop(0, ept // ECHUNK, chunk, ())
    plsc.subcore_barrier()
    @pl.when(sid == 0)
    def _():
        pltpu.sync_copy(a_sh, a_out.at[pl.ds(cid * ns, ns)])


def _sc_prep2(w, d0, d1, src_pad, dst_pad):
    ns = w.shape[0]
    ept = src_pad.shape[0] // NW
    f = pl.kernel(
        functools.partial(_prep2_body, ns, ept),
        out_type=jax.ShapeDtypeStruct((NCORE * ns,), F32),
        mesh=_sc_mesh(),
        scratch_types=[
            pltpu.VMEM_SHARED((ns,), F32),
            pltpu.VMEM((ECHUNK,), I32),
            pltpu.VMEM((ECHUNK,), I32),
            pltpu.VMEM((ECHUNK,), F32),
            pltpu.VMEM((ECHUNK,), F32),
            pltpu.VMEM((ECHUNK,), F32),
            pltpu.VMEM((ECHUNK,), F32),
            pltpu.VMEM((ZF,), F32),
            pltpu.SemaphoreType.DMA,
        ],
    )
    a_flat = f(w, d0, d1, src_pad, dst_pad)
    return a_flat[:ns], a_flat[ns:]


def _conv_body(ns, ept, ec, with_pos, rows_by_dst, x_hbm, px_hbm, py_hbm,
               pz_hbm, w_hbm, d0_hbm, d1_hbm, a0_hbm, a1_hbm, src_hbm,
               dst_hbm, h_out, px_o, py_o, pz_o,
               h_sh, px_sh, py_sh, pz_sh,
               x_a, si_a, di_a, wg_a, g0_a, g1_a, a0_a, a1_a, pxg_a, pyg_a,
               pzg_a, x_c, si_c, di_c, wg_c, g0_c, g1_c, a0_c, a1_c, pxg_c,
               pyg_c, pzg_c, zbuf, zflat, gsem_a, gsem_c, ssem_a, ssem_c):
    cid = lax.axis_index("c")
    sid = lax.axis_index("s")
    wid = _wid()
    nr = ns // NSUB

    bufA = (x_a, si_a, di_a, wg_a, g0_a, g1_a, a0_a, a1_a, pxg_a, pyg_a,
            pzg_a)
    bufB = (x_c, si_c, di_c, wg_c, g0_c, g1_c, a0_c, a1_c, pxg_c, pyg_c,
            pzg_c)

    _fill_rows(zbuf, 0.0)
    _zero_rows(h_sh, sid * nr, nr, zbuf)
    @pl.when(sid == 0)
    def _():
        _fill_flat(zflat, 0.0)
        if with_pos:
            _zero_flat_sh(px_sh, zflat)
            _zero_flat_sh(py_sh, zflat)
            _zero_flat_sh(pz_sh, zflat)
    plsc.subcore_barrier()
    base = wid * ept
    nc = ept // ec

    def stage_idx(jc, buf):
        off = base + jc * ec
        d1_ = pltpu.async_copy(src_hbm.at[pl.ds(off, ec)], buf[1], gsem_a)
        d2_ = pltpu.async_copy(dst_hbm.at[pl.ds(off, ec)], buf[2], gsem_a)
        d1_.wait()
        d2_.wait()

    def gather_args(buf):
        (x_b, si_b, di_b, wg, g0, g1, a0g, a1g, pxg, pyg, pzg) = buf
        gi = di_b if rows_by_dst else si_b
        args = [(x_hbm.at[gi], x_b), (w_hbm.at[si_b], wg),
                (d0_hbm.at[si_b], g0), (d1_hbm.at[si_b], g1),
                (a0_hbm.at[di_b], a0g), (a1_hbm.at[di_b], a1g)]
        if with_pos:
            args += [(px_hbm.at[si_b], pxg), (py_hbm.at[si_b], pyg),
                     (pz_hbm.at[si_b], pzg)]
        return args

    def fire_gathers(buf, sem):
        for s_, d_ in gather_args(buf):
            pltpu.async_copy(s_, d_, sem)

    def drain_gathers(buf, sem):
        for s_, d_ in gather_args(buf):
            pltpu.make_async_copy(s_, d_, sem).wait()

    def scatter_args(buf):
        (x_b, si_b, di_b, wg, g0, g1, a0g, a1g, pxg, pyg, pzg) = buf
        sc = si_b if rows_by_dst else di_b
        args = [(x_b, h_sh.at[sc])]
        if with_pos:
            args += [(pxg, px_sh.at[sc]), (pyg, py_sh.at[sc]),
                     (pzg, pz_sh.at[sc])]
        return args

    def fire_scatters(buf, sem):
        for s_, d_ in scatter_args(buf):
            pltpu.async_copy(s_, d_, sem, add=True)

    def drain_scatters(buf, sem):
        for s_, d_ in scatter_args(buf):
            pltpu.make_async_copy(s_, d_, sem).wait()

    def compute(buf):
        (x_b, si_b, di_b, wg, g0, g1, a0g, a1g, pxg, pyg, pzg) = buf

        def group(g, _):
            gb = g * 16
            cs16 = pl.ds(gb, 16)
            deg = jnp.maximum(g0[cs16] + g1[cs16], 1.0)
            ecv = (wg[cs16] / deg) / (a0g[cs16] + a1g[cs16] + 1e-12)
            if with_pos:
                pxg[cs16] = ecv * pxg[cs16]
                pyg[cs16] = ecv * pyg[cs16]
                pzg[cs16] = ecv * pzg[cs16]
            for e in range(16):
                row = gb + e
                eb = _bcast_lane(ecv, e)
                for k in range(8):
                    cs = pl.ds(16 * k, 16)
                    x_b[row, cs] = eb * x_b[row, cs]
            return ()
        lax.fori_loop(0, ec // 16, group, ())

    stage_idx(0, bufA)
    fire_gathers(bufA, gsem_a)

    def pair(t, _):
        j0 = 2 * t
        drain_gathers(bufA, gsem_a)
        @pl.when(t > 0)
        def _():
            drain_scatters(bufB, ssem_c)
        stage_idx(j0 + 1, bufB)
        fire_gathers(bufB, gsem_c)
        compute(bufA)
        fire_scatters(bufA, ssem_a)
        drain_gathers(bufB, gsem_c)
        drain_scatters(bufA, ssem_a)
        @pl.when(j0 + 2 < nc)
        def _():
            stage_idx(j0 + 2, bufA)
            fire_gathers(bufA, gsem_a)
        compute(bufB)
        fire_scatters(bufB, ssem_c)
        return ()
    lax.fori_loop(0, nc // 2, pair, ())
    drain_scatters(bufB, ssem_c)
    plsc.subcore_barrier()

    r0 = sid * nr
    pltpu.sync_copy(h_sh.at[pl.ds(r0, nr)], h_out.at[cid, pl.ds(r0, nr)])
    if with_pos:
        @pl.when(sid == 0)
        def _():
            pltpu.sync_copy(px_sh, px_o.at[pl.ds(cid * ns, ns)])
            pltpu.sync_copy(py_sh, py_o.at[pl.ds(cid * ns, ns)])
            pltpu.sync_copy(pz_sh, pz_o.at[pl.ds(cid * ns, ns)])


def _sc_conv(x, pxyz, w, d0, d1, a0, a1, src_pad, dst_pad, with_pos,
             rows_by_dst, ec=64):
    ns = x.shape[0]
    ept = src_pad.shape[0] // NW
    out_type = [jax.ShapeDtypeStruct((NCORE, ns, LD), F32)]
    out_type += [jax.ShapeDtypeStruct((NCORE * ns,), F32)] * 3
    scr = (
        [pltpu.VMEM_SHARED((ns, LD), F32)]
        + [pltpu.VMEM_SHARED((ns,), F32)] * 3
        + ([pltpu.VMEM((ec, LD), F32), pltpu.VMEM((ec,), I32),
            pltpu.VMEM((ec,), I32)] + [pltpu.VMEM((ec,), F32)] * 8) * 2
        + [pltpu.VMEM((16, LD), F32), pltpu.VMEM((ZF,), F32),
           pltpu.SemaphoreType.DMA, pltpu.SemaphoreType.DMA,
           pltpu.SemaphoreType.DMA, pltpu.SemaphoreType.DMA]
    )
    f = pl.kernel(
        functools.partial(_conv_body, ns, ept, ec, with_pos, rows_by_dst),
        out_type=out_type,
        mesh=_sc_mesh(),
        scratch_types=scr,
    )
    px, py, pz = pxyz if with_pos else (w, w, w)
    return f(x, px, py, pz, w, d0, d1, a0, a1, src_pad, dst_pad)


GCH = 80


def _compact_body(ns_out, hp0_hbm, hp1_hbm, px0, px1, py0, py1, pz0, pz1,
                  aw0, aw1, mid_hbm, h_out, px_o, py_o, pz_o, w_o,
                  mid_b, r0_b, r1_b, s0_b, s1_b):
    wid = _wid()
    nrw = ns_out // NW
    base = wid * nrw
    for j0 in range(0, nrw, GCH):
        off = base + j0
        pltpu.sync_copy(mid_hbm.at[pl.ds(off, GCH)], mid_b)
        pltpu.sync_copy(hp0_hbm.at[mid_b], r0_b)
        pltpu.sync_copy(hp1_hbm.at[mid_b], r1_b)

        def add_rows(i, _):
            for k in range(8):
                cs = pl.ds(16 * k, 16)
                r0_b[i, cs] = r0_b[i, cs] + r1_b[i, cs]
            return ()
        lax.fori_loop(0, GCH, add_rows, ())
        pltpu.sync_copy(r0_b, h_out.at[pl.ds(off, GCH)])

        for (a_, b_, o_, eps) in ((px0, px1, px_o, 0.0),
                                  (py0, py1, py_o, 0.0),
                                  (pz0, pz1, pz_o, 0.0),
                                  (aw0, aw1, w_o, 1e-12)):
            pltpu.sync_copy(a_.at[mid_b], s0_b)
            pltpu.sync_copy(b_.at[mid_b], s1_b)

            def add_s(i, _):
                cs = pl.ds(16 * i, 16)
                s0_b[cs] = s0_b[cs] + s1_b[cs] + eps
                return ()
            lax.fori_loop(0, GCH // 16, add_s, ())
            pltpu.sync_copy(s0_b, o_.at[pl.ds(off, GCH)])


def _sc_compact(hp, pxp, pyp, pzp, awp, mid_pad):
    ns_out = mid_pad.shape[0]
    f = pl.kernel(
        functools.partial(_compact_body, ns_out),
        out_type=[
            jax.ShapeDtypeStruct((ns_out, LD), F32),
            jax.ShapeDtypeStruct((ns_out,), F32),
            jax.ShapeDtypeStruct((ns_out,), F32),
            jax.ShapeDtypeStruct((ns_out,), F32),
            jax.ShapeDtypeStruct((ns_out,), F32),
        ],
        mesh=_sc_mesh(),
        scratch_types=[
            pltpu.VMEM((GCH,), I32),
            pltpu.VMEM((GCH, LD), F32),
            pltpu.VMEM((GCH, LD), F32),
            pltpu.VMEM((GCH,), F32),
            pltpu.VMEM((GCH,), F32),
        ],
    )
    return f(hp[0], hp[1], pxp[0], pxp[1], pyp[0], pyp[1], pzp[0], pzp[1],
             awp[0], awp[1], mid_pad)


def _unpool_body(ns_out, hse_hbm, sel_hbm, u_out, sel_b, r_b):
    wid = _wid()
    nrw = ns_out // NW
    base = wid * nrw
    for j0 in range(0, nrw, GCH):
        off = base + j0
        pltpu.sync_copy(sel_hbm.at[pl.ds(off, GCH)], sel_b)
        pltpu.sync_copy(hse_hbm.at[sel_b], r_b)
        pltpu.sync_copy(r_b, u_out.at[pl.ds(off, GCH)])


def _sc_unpool(h_small_ext, sel_pad):
    ns_out = sel_pad.shape[0]
    f = pl.kernel(
        functools.partial(_unpool_body, ns_out),
        out_type=jax.ShapeDtypeStruct((ns_out, LD), F32),
        mesh=_sc_mesh(),
        scratch_types=[
            pltpu.VMEM((GCH,), I32),
            pltpu.VMEM((GCH, LD), F32),
        ],
    )
    return f(h_small_ext, sel_pad)


def _pre_body(h_ref, pos_ref, wa_ref, wb_ref, wp_ref, b1_ref, u_ref, v_ref):
    h = h_ref[...]
    posp = pos_ref[...] @ wp_ref[...]
    u_ref[...] = h @ wa_ref[...] + posp + b1_ref[...]
    v_ref[...] = h @ wb_ref[...] - posp


def _tc_pre(h, posr, wa, wb, wp, b1, blk=512):
    n = h.shape[0]
    return pl.pallas_call(
        _pre_body,
        grid=(_ceil(n, blk),),
        in_specs=[
            pl.BlockSpec((blk, LD), lambda i: (i, 0)),
            pl.BlockSpec((blk, 16), lambda i: (i, 0)),
            pl.BlockSpec((LD, LD), lambda i: (0, 0)),
            pl.BlockSpec((LD, LD), lambda i: (0, 0)),
            pl.BlockSpec((16, LD), lambda i: (0, 0)),
            pl.BlockSpec((1, LD), lambda i: (0, 0)),
        ],
        out_specs=[
            pl.BlockSpec((blk, LD), lambda i: (i, 0)),
            pl.BlockSpec((blk, LD), lambda i: (i, 0)),
        ],
        out_shape=[
            jax.ShapeDtypeStruct((n, LD), F32),
            jax.ShapeDtypeStruct((n, LD), F32),
        ],
    )(h, posr, wa, wb, wp, b1)


def _node_body(h_ref, s0_ref, s1_ref, d0_ref, d1_ref, w2e_ref, b2e_ref,
               wna_ref, wnb_ref, bn1_ref, wn2_ref, bn2_ref, res_ref, o_ref):
    h = h_ref[...]
    deg = d0_ref[...] + d1_ref[...]
    aggr = (s0_ref[0] + s1_ref[0]) @ w2e_ref[...] + deg * b2e_ref[...]
    z = jnp.maximum(h @ wna_ref[...] + aggr @ wnb_ref[...] + bn1_ref[...],
                    0.0)
    o_ref[...] = h + z @ wn2_ref[...] + bn2_ref[...] + res_ref[...]


def _tc_node(h, s_part, d0, d1, w2e, b2e, wna, wnb, bn1, wn2, bn2, res,
             blk=512):
    n = h.shape[0]
    return pl.pallas_call(
        _node_body,
        grid=(_ceil(n, blk),),
        in_specs=[
            pl.BlockSpec((blk, LD), lambda i: (i, 0)),
            pl.BlockSpec((1, blk, LD), lambda i: (0, i, 0)),
            pl.BlockSpec((1, blk, LD), lambda i: (1, i, 0)),
            pl.BlockSpec((blk, 1), lambda i: (i, 0)),
            pl.BlockSpec((blk, 1), lambda i: (i, 0)),
            pl.BlockSpec((LD, LD), lambda i: (0, 0)),
            pl.BlockSpec((1, LD), lambda i: (0, 0)),
            pl.BlockSpec((LD, LD), lambda i: (0, 0)),
            pl.BlockSpec((LD, LD), lambda i: (0, 0)),
            pl.BlockSpec((1, LD), lambda i: (0, 0)),
            pl.BlockSpec((LD, LD), lambda i: (0, 0)),
            pl.BlockSpec((1, LD), lambda i: (0, 0)),
            pl.BlockSpec((blk, LD), lambda i: (i, 0)),
        ],
        out_specs=pl.BlockSpec((blk, LD), lambda i: (i, 0)),
        out_shape=jax.ShapeDtypeStruct((n, LD), F32),
    )(h, s_part, s_part, d0[:, None], d1[:, None], w2e, b2e, wna, wnb, bn1,
      wn2, bn2, res)


def _add_body(a_ref, b_ref, o_ref):
    o_ref[...] = a_ref[0] + b_ref[0]


def _tc_add2(parts, blk=512):
    n = parts.shape[1]
    return pl.pallas_call(
        _add_body,
        grid=(_ceil(n, blk),),
        in_specs=[
            pl.BlockSpec((1, blk, LD), lambda i: (0, i, 0)),
            pl.BlockSpec((1, blk, LD), lambda i: (1, i, 0)),
        ],
        out_specs=pl.BlockSpec((blk, LD), lambda i: (i, 0)),
        out_shape=jax.ShapeDtypeStruct((n, LD), F32),
    )(parts, parts)


def _split_gmp_params(p):
    w1e = p["edge"]["W1"]
    wa = w1e[:LD]
    wb = w1e[LD:2 * LD]
    wp = jnp.pad(w1e[2 * LD:2 * LD + 3], ((0, 13), (0, 0)))
    wd = w1e[2 * LD + 3]
    b1 = p["edge"]["b1"][None, :]
    w2e = p["edge"]["W2"]
    b2e = p["edge"]["b2"][None, :]
    wn1 = p["node"]["W1"]
    return (wa, wb, wp, wd, b1, w2e, b2e, wn1[:LD], wn1[LD:],
            p["node"]["b1"][None, :], p["node"]["W2"],
            p["node"]["b2"][None, :])


def _pad_edges(src, dst, n):
    e = src.shape[0]
    g = NW * ECHUNK * 2
    epad = _ceil(e, g) * g
    fill = jnp.full((epad - e,), n, I32)
    return jnp.concatenate([src, fill]), jnp.concatenate([dst, fill])


def _gmp_sc(p, h_pad, posr, pxyz, srcp, dstp, res=None):
    (wa, wb, wp, wd, b1, w2e, b2e, wna, wnb, bn1, wn2, bn2) = \
        _split_gmp_params(p)
    u, v = _tc_pre(h_pad, posr, wa, wb, wp, b1)
    s_part, dd0, dd1 = _sc_edge(u, v, pxyz, srcp, dstp, wd)
    if res is None:
        res = jnp.zeros_like(h_pad)
    return _tc_node(h_pad, s_part, dd0, dd1, w2e, b2e, wna, wnb, bn1, wn2,
                    bn2, res)


def _pad_rows(x, ns):
    return jnp.pad(x, ((0, ns - x.shape[0]), (0, 0)))


def _pad_flat(x, ns):
    return jnp.pad(x, (0, ns - x.shape[0]))


def kernel(h, pos, params, m_gs_0, m_gs_1, m_gs_2, m_ids_0, m_ids_1):
    m_gs = [m_gs_0, m_gs_1, m_gs_2]
    m_ids = [m_ids_0, m_ids_1]
    nlist = [10000, 5000, 2500]

    ns0 = NS_TAB[10000]
    h_pad = _pad_rows(h, ns0)
    posr = jnp.pad(pos, ((0, ns0 - pos.shape[0]), (0, 13)))
    pxyz = (posr[:, 0], posr[:, 1], posr[:, 2])
    w = _pad_flat(jnp.ones((10000,), F32), ns0)

    down_hs, down_geo, down_sc, pads = [], [], [], []
    for i in range(2):
        n = nlist[i]
        ns = NS_TAB[n]
        nsn = NS_TAB[nlist[i + 1]]
        srcp, dstp = _pad_edges(m_gs[i][0], m_gs[i][1], n)
        pads.append((srcp, dstp))
        h_pad = _gmp_sc(params["down"][i], h_pad, posr, pxyz, srcp, dstp)
        down_hs.append(h_pad)
        down_geo.append((posr, pxyz))
        d0, d1 = _sc_prep1(srcp, ns)
        a0, a1 = _sc_prep2(w, d0, d1, srcp, dstp)
        down_sc.append((w, d0, d1, a0, a1))
        hp, pxf, pyf, pzf = _sc_conv(
            h_pad, pxyz, w, d0, d1, a0, a1, srcp, dstp, True, False)
        mid_pad = _pad_flat(m_ids[i], nsn)
        h_pad, px, py, pz, w = _sc_compact(
            hp, (pxf[:ns], pxf[ns:]), (pyf[:ns], pyf[ns:]),
            (pzf[:ns], pzf[ns:]), (a0, a1), mid_pad)
        posr = jnp.concatenate(
            [px[:, None], py[:, None], pz[:, None],
             jnp.zeros((nsn, 13), F32)], axis=1)
        pxyz = (px, py, pz)

    n2 = nlist[2]
    ns2 = NS_TAB[n2]
    srcp, dstp = _pad_edges(m_gs[2][0], m_gs[2][1], n2)
    h_pad = _gmp_sc(params["bottom"], h_pad, posr, pxyz, srcp, dstp)

    for i in range(2):
        li = 1 - i
        n = nlist[li]
        ns = NS_TAB[n]
        ns_small = NS_TAB[nlist[li + 1]]
        srcp, dstp = pads[li]
        posr, pxyz = down_geo[li]
        w_l, d0, d1, a0, a1 = down_sc[li]
        hse = jnp.concatenate([h_pad, jnp.zeros((LD, LD), F32)], axis=0)
        zrow = ns_small
        sel = jnp.full((ns,), zrow, I32).at[m_ids[li]].set(
            jnp.arange(nlist[li + 1], dtype=I32))
        u_arr = _sc_unpool(hse, sel)
        hu = _sc_conv(u_arr, None, w_l, d0, d1, a0, a1, srcp, dstp,
                      False, True)[0]
        h_uc = _tc_add2(hu)
        h_pad = _gmp_sc(params["up"][i], h_uc, posr, pxyz, srcp, dstp,
                        res=down_hs[li])
    return h_pad[:10000]


# speedup vs baseline: 3.5681x; 1.0195x over previous
import functools
import jax
import jax.numpy as jnp
from jax import lax
from jax.experimental import pallas as pl
from jax.experimental.pallas import tpu as pltpu
from jax.experimental.pallas import tpu_sc as plsc

LD = 128
NCORE = 2
NSUB = 16
NW = NCORE * NSUB
ECHUNK = 128
F32 = jnp.float32
I32 = jnp.int32

NS_TAB = {10000: 10240, 5000: 5120, 2500: 2560}


def _ceil(a, b):
    return (a + b - 1) // b


def _sc_mesh():
    return plsc.VectorSubcoreMesh(core_axis_name="c", subcore_axis_name="s",
                                  num_cores=NCORE, num_subcores=NSUB)


def _wid():
    return lax.axis_index("s") * NCORE + lax.axis_index("c")


def _fill_rows(buf_ref, val):
    def zrow(i, _):
        for k in range(buf_ref.shape[1] // 16):
            buf_ref[i, pl.ds(16 * k, 16)] = jnp.full((16,), val, F32)
        return ()
    lax.fori_loop(0, buf_ref.shape[0], zrow, ())


def _fill_flat(buf_ref, val):
    def zi(i, _):
        buf_ref[pl.ds(16 * i, 16)] = jnp.full((16,), val, F32)
        return ()
    lax.fori_loop(0, buf_ref.shape[0] // 16, zi, ())


ZF = 512


def _zero_flat_sh(sh_ref, zflat_ref):
    n = sh_ref.shape[0]
    def cp(j, _):
        pltpu.sync_copy(zflat_ref, sh_ref.at[pl.ds(j * ZF, ZF)])
        return ()
    lax.fori_loop(0, n // ZF, cp, ())


def _zero_rows(sh_ref, row0, nrows, zbuf_ref):
    zr = zbuf_ref.shape[0]
    def cp(j, _):
        pltpu.sync_copy(zbuf_ref, sh_ref.at[pl.ds(row0 + j * zr, zr)])
        return ()
    lax.fori_loop(0, nrows // zr, cp, ())


def _rsqrt_nr(d2v):
    i0 = lax.bitcast_convert_type(d2v, I32)
    y = lax.bitcast_convert_type(0x5F3759DF - (i0 >> 1), F32)
    for _ in range(3):
        y = y * (1.5 - 0.5 * d2v * y * y)
    return y


def _bcast_lane(vec, e):
    return vec.at[jnp.full((16,), e, I32)].get(mode="promise_in_bounds")


def _edge_body(ns, ept, ec, u_hbm, v_hbm, px_hbm, py_hbm, pz_hbm, src_hbm,
               dst_hbm, wd_hbm, s_out, d_out, s_sh, d_sh,
               u_a, v_a, si_a, di_a, pxs_a, pys_a, pzs_a, pxd_a, pyd_a,
               pzd_a, u_c, v_c, si_c, di_c, pxs_c, pys_c, pzs_c, pxd_c,
               pyd_c, pzd_c, wd_b, ones_b, zbuf, zflat, gsem_a, gsem_c,
               ssem_a, ssem_c):
    cid = lax.axis_index("c")
    sid = lax.axis_index("s")
    wid = _wid()
    nr = ns // NSUB

    bufA = (u_a, v_a, si_a, di_a, pxs_a, pys_a, pzs_a, pxd_a, pyd_a, pzd_a)
    bufB = (u_c, v_c, si_c, di_c, pxs_c, pys_c, pzs_c, pxd_c, pyd_c, pzd_c)

    _fill_rows(zbuf, 0.0)
    _zero_rows(s_sh, sid * nr, nr, zbuf)
    _fill_flat(ones_b, 1.0)
    @pl.when(sid == 0)
    def _():
        _fill_flat(zflat, 0.0)
        _zero_flat_sh(d_sh, zflat)
    pltpu.sync_copy(wd_hbm, wd_b)
    plsc.subcore_barrier()

    wdv = [wd_b[pl.ds(16 * k, 16)] for k in range(8)]
    base = wid * ept
    nc = ept // ec

    def stage_idx(jc, buf):
        off = base + jc * ec
        d1_ = pltpu.async_copy(src_hbm.at[pl.ds(off, ec)], buf[2], gsem_a)
        d2_ = pltpu.async_copy(dst_hbm.at[pl.ds(off, ec)], buf[3], gsem_a)
        d1_.wait()
        d2_.wait()

    def gather_args(buf):
        u_b, v_b, si_b, di_b, pxs, pys, pzs, pxd, pyd, pzd = buf
        return [(u_hbm.at[si_b], u_b), (v_hbm.at[di_b], v_b),
                (px_hbm.at[si_b], pxs), (py_hbm.at[si_b], pys),
                (pz_hbm.at[si_b], pzs), (px_hbm.at[di_b], pxd),
                (py_hbm.at[di_b], pyd), (pz_hbm.at[di_b], pzd)]

    def fire_gathers(buf, sem):
        for s_, d_ in gather_args(buf):
            pltpu.async_copy(s_, d_, sem)

    def drain_gathers(buf, sem):
        for s_, d_ in gather_args(buf):
            pltpu.make_async_copy(s_, d_, sem).wait()

    def fire_scatters(buf, sem):
        pltpu.async_copy(buf[0], s_sh.at[buf[3]], sem, add=True)
        pltpu.async_copy(ones_b, d_sh.at[buf[3]], sem, add=True)

    def drain_scatters(buf, sem):
        pltpu.make_async_copy(buf[0], s_sh.at[buf[3]], sem).wait()
        pltpu.make_async_copy(ones_b, d_sh.at[buf[3]], sem).wait()

    def compute(buf):
        u_b, v_b, si_b, di_b, pxs, pys, pzs, pxd, pyd, pzd = buf

        def group(g, _):
            gb = g * 16
            cs16 = pl.ds(gb, 16)
            dx = pxs[cs16] - pxd[cs16]
            dy = pys[cs16] - pyd[cs16]
            dz = pzs[cs16] - pzd[cs16]
            d2v = dx * dx + dy * dy + dz * dz + 1e-12
            dist = d2v * _rsqrt_nr(d2v)
            for e in range(16):
                row = gb + e
                db = _bcast_lane(dist, e)
                for k in range(8):
                    cs = pl.ds(16 * k, 16)
                    pre = u_b[row, cs] + v_b[row, cs] + db * wdv[k]
                    u_b[row, cs] = jnp.maximum(pre, 0.0)
            return ()
        lax.fori_loop(0, ec // 16, group, ())

    stage_idx(0, bufA)
    fire_gathers(bufA, gsem_a)

    def pair(t, _):
        j0 = 2 * t
        drain_gathers(bufA, gsem_a)
        @pl.when(t > 0)
        def _():
            drain_scatters(bufB, ssem_c)
        stage_idx(j0 + 1, bufB)
        fire_gathers(bufB, gsem_c)
        compute(bufA)
        fire_scatters(bufA, ssem_a)
        drain_gathers(bufB, gsem_c)
        drain_scatters(bufA, ssem_a)
        @pl.when(j0 + 2 < nc)
        def _():
            stage_idx(j0 + 2, bufA)
            fire_gathers(bufA, gsem_a)
        compute(bufB)
        fire_scatters(bufB, ssem_c)
        return ()
    lax.fori_loop(0, nc // 2, pair, ())
    drain_scatters(bufB, ssem_c)
    plsc.subcore_barrier()

    r0 = sid * nr
    pltpu.sync_copy(s_sh.at[pl.ds(r0, nr)], s_out.at[cid, pl.ds(r0, nr)])
    @pl.when(sid == 0)
    def _():
        pltpu.sync_copy(d_sh, d_out.at[pl.ds(cid * ns, ns)])


def _sc_edge(u, v, pxyz, src_pad, dst_pad, wd, ec=64):
    ns = u.shape[0]
    ept = src_pad.shape[0] // NW
    f = pl.kernel(
        functools.partial(_edge_body, ns, ept, ec),
        out_type=[
            jax.ShapeDtypeStruct((NCORE, ns, LD), F32),
            jax.ShapeDtypeStruct((NCORE * ns,), F32),
        ],
        mesh=_sc_mesh(),
        scratch_types=(
            [pltpu.VMEM_SHARED((ns, LD), F32),
             pltpu.VMEM_SHARED((ns,), F32)] +
            ([pltpu.VMEM((ec, LD), F32), pltpu.VMEM((ec, LD), F32),
              pltpu.VMEM((ec,), I32), pltpu.VMEM((ec,), I32)]
             + [pltpu.VMEM((ec,), F32)] * 6) * 2
            + [pltpu.VMEM((LD,), F32), pltpu.VMEM((ec,), F32),
               pltpu.VMEM((16, LD), F32), pltpu.VMEM((ZF,), F32),
               pltpu.SemaphoreType.DMA, pltpu.SemaphoreType.DMA,
               pltpu.SemaphoreType.DMA, pltpu.SemaphoreType.DMA]
        ),
    )
    s_part, d_flat = f(u, v, pxyz[0], pxyz[1], pxyz[2], src_pad, dst_pad, wd)
    return s_part, d_flat[:ns], d_flat[ns:]


def _prep1_body(ns, ept, src_hbm, d_out, d_sh, si_b, ones_b, zflat):
    cid = lax.axis_index("c")
    sid = lax.axis_index("s")
    wid = _wid()
    _fill_flat(ones_b, 1.0)
    @pl.when(sid == 0)
    def _():
        _fill_flat(zflat, 0.0)
        _zero_flat_sh(d_sh, zflat)
    plsc.subcore_barrier()
    base = wid * ept

    def chunk(jc, _):
        pltpu.sync_copy(src_hbm.at[pl.ds(base + jc * ECHUNK, ECHUNK)], si_b)
        pltpu.sync_copy(ones_b, d_sh.at[si_b], add=True)
        return ()
    lax.fori_loop(0, ept // ECHUNK, chunk, ())
    plsc.subcore_barrier()
    @pl.when(sid == 0)
    def _():
        pltpu.sync_copy(d_sh, d_out.at[pl.ds(cid * ns, ns)])


def _sc_prep1(src_pad, ns):
    ept = src_pad.shape[0] // NW
    f = pl.kernel(
        functools.partial(_prep1_body, ns, ept),
        out_type=jax.ShapeDtypeStruct((NCORE * ns,), F32),
        mesh=_sc_mesh(),
        scratch_types=[
            pltpu.VMEM_SHARED((ns,), F32),
            pltpu.VMEM((ECHUNK,), I32),
            pltpu.VMEM((ECHUNK,), F32),
            pltpu.VMEM((ZF,), F32),
        ],
    )
    d_flat = f(src_pad)
    return d_flat[:ns], d_flat[ns:]


def _prep2_body(ns, ept, w_hbm, d0_hbm, d1_hbm, src_hbm, dst_hbm,
                a_out, a_sh, si_b, di_b, wg, g0, g1, wts_b, zflat, sem):
    cid = lax.axis_index("c")
    sid = lax.axis_index("s")
    wid = _wid()
    @pl.when(sid == 0)
    def _():
        _fill_flat(zflat, 0.0)
        _zero_flat_sh(a_sh, zflat)
    plsc.subcore_barrier()
    base = wid * ept

    def chunk(jc, _):
        off = base + jc * ECHUNK
        d1_ = pltpu.async_copy(src_hbm.at[pl.ds(off, ECHUNK)], si_b, sem)
        d2_ = pltpu.async_copy(dst_hbm.at[pl.ds(off, ECHUNK)], di_b, sem)
        d1_.wait()
        d2_.wait()
        descs = [
            pltpu.async_copy(w_hbm.at[si_b], wg, sem),
            pltpu.async_copy(d0_hbm.at[si_b], g0, sem),
            pltpu.async_copy(d1_hbm.at[si_b], g1, sem),
        ]
        for d_ in descs:
            d_.wait()

        def group(g, _):
            cs = pl.ds(g * 16, 16)
            deg = jnp.maximum(g0[cs] + g1[cs], 1.0)
            wts_b[cs] = wg[cs] / deg
            return ()
        lax.fori_loop(0, ECHUNK // 16, group, ())
        pltpu.sync_copy(wts_b, a_sh.at[di_b], add=True)
        return ()
    lax.fori_loop(0, ept // ECHUNK, chunk, ())
    plsc.subcore_barrier()
    @pl.when(sid == 0)
    def _():
        pltpu.sync_copy(a_sh, a_out.at[pl.ds(cid * ns, ns)])


def _sc_prep2(w, d0, d1, src_pad, dst_pad):
    ns = w.shape[0]
    ept = src_pad.shape[0] // NW
    f = pl.kernel(
        functools.partial(_prep2_body, ns, ept),
        out_type=jax.ShapeDtypeStruct((NCORE * ns,), F32),
        mesh=_sc_mesh(),
        scratch_types=[
            pltpu.VMEM_SHARED((ns,), F32),
            pltpu.VMEM((ECHUNK,), I32),
            pltpu.VMEM((ECHUNK,), I32),
            pltpu.VMEM((ECHUNK,), F32),
            pltpu.VMEM((ECHUNK,), F32),
            pltpu.VMEM((ECHUNK,), F32),
            pltpu.VMEM((ECHUNK,), F32),
            pltpu.VMEM((ZF,), F32),
            pltpu.SemaphoreType.DMA,
        ],
    )
    a_flat = f(w, d0, d1, src_pad, dst_pad)
    return a_flat[:ns], a_flat[ns:]


def _conv_body(ns, ept, ec, with_pos, rows_by_dst, x_hbm, px_hbm, py_hbm,
               pz_hbm, w_hbm, d0_hbm, d1_hbm, a0_hbm, a1_hbm, src_hbm,
               dst_hbm, h_out, px_o, py_o, pz_o,
               h_sh, px_sh, py_sh, pz_sh,
               x_a, si_a, di_a, wg_a, g0_a, g1_a, a0_a, a1_a, pxg_a, pyg_a,
               pzg_a, x_c, si_c, di_c, wg_c, g0_c, g1_c, a0_c, a1_c, pxg_c,
               pyg_c, pzg_c, zbuf, zflat, gsem_a, gsem_c, ssem_a, ssem_c):
    cid = lax.axis_index("c")
    sid = lax.axis_index("s")
    wid = _wid()
    nr = ns // NSUB

    bufA = (x_a, si_a, di_a, wg_a, g0_a, g1_a, a0_a, a1_a, pxg_a, pyg_a,
            pzg_a)
    bufB = (x_c, si_c, di_c, wg_c, g0_c, g1_c, a0_c, a1_c, pxg_c, pyg_c,
            pzg_c)

    _fill_rows(zbuf, 0.0)
    _zero_rows(h_sh, sid * nr, nr, zbuf)
    @pl.when(sid == 0)
    def _():
        _fill_flat(zflat, 0.0)
        if with_pos:
            _zero_flat_sh(px_sh, zflat)
            _zero_flat_sh(py_sh, zflat)
            _zero_flat_sh(pz_sh, zflat)
    plsc.subcore_barrier()
    base = wid * ept
    nc = ept // ec

    def stage_idx(jc, buf):
        off = base + jc * ec
        d1_ = pltpu.async_copy(src_hbm.at[pl.ds(off, ec)], buf[1], gsem_a)
        d2_ = pltpu.async_copy(dst_hbm.at[pl.ds(off, ec)], buf[2], gsem_a)
        d1_.wait()
        d2_.wait()

    def gather_args(buf):
        (x_b, si_b, di_b, wg, g0, g1, a0g, a1g, pxg, pyg, pzg) = buf
        gi = di_b if rows_by_dst else si_b
        args = [(x_hbm.at[gi], x_b), (w_hbm.at[si_b], wg),
                (d0_hbm.at[si_b], g0), (d1_hbm.at[si_b], g1),
                (a0_hbm.at[di_b], a0g), (a1_hbm.at[di_b], a1g)]
        if with_pos:
            args += [(px_hbm.at[si_b], pxg), (py_hbm.at[si_b], pyg),
                     (pz_hbm.at[si_b], pzg)]
        return args

    def fire_gathers(buf, sem):
        for s_, d_ in gather_args(buf):
            pltpu.async_copy(s_, d_, sem)

    def drain_gathers(buf, sem):
        for s_, d_ in gather_args(buf):
            pltpu.make_async_copy(s_, d_, sem).wait()

    def scatter_args(buf):
        (x_b, si_b, di_b, wg, g0, g1, a0g, a1g, pxg, pyg, pzg) = buf
        sc = si_b if rows_by_dst else di_b
        args = [(x_b, h_sh.at[sc])]
        if with_pos:
            args += [(pxg, px_sh.at[sc]), (pyg, py_sh.at[sc]),
                     (pzg, pz_sh.at[sc])]
        return args

    def fire_scatters(buf, sem):
        for s_, d_ in scatter_args(buf):
            pltpu.async_copy(s_, d_, sem, add=True)

    def drain_scatters(buf, sem):
        for s_, d_ in scatter_args(buf):
            pltpu.make_async_copy(s_, d_, sem).wait()

    def compute(buf):
        (x_b, si_b, di_b, wg, g0, g1, a0g, a1g, pxg, pyg, pzg) = buf

        def group(g, _):
            gb = g * 16
            cs16 = pl.ds(gb, 16)
            deg = jnp.maximum(g0[cs16] + g1[cs16], 1.0)
            ecv = (wg[cs16] / deg) / (a0g[cs16] + a1g[cs16] + 1e-12)
            if with_pos:
                pxg[cs16] = ecv * pxg[cs16]
                pyg[cs16] = ecv * pyg[cs16]
                pzg[cs16] = ecv * pzg[cs16]
            for e in range(16):
                row = gb + e
                eb = _bcast_lane(ecv, e)
                for k in range(8):
                    cs = pl.ds(16 * k, 16)
                    x_b[row, cs] = eb * x_b[row, cs]
            return ()
        lax.fori_loop(0, ec // 16, group, ())

    stage_idx(0, bufA)
    fire_gathers(bufA, gsem_a)

    def pair(t, _):
        j0 = 2 * t
        drain_gathers(bufA, gsem_a)
        @pl.when(t > 0)
        def _():
            drain_scatters(bufB, ssem_c)
        stage_idx(j0 + 1, bufB)
        fire_gathers(bufB, gsem_c)
        compute(bufA)
        fire_scatters(bufA, ssem_a)
        drain_gathers(bufB, gsem_c)
        drain_scatters(bufA, ssem_a)
        @pl.when(j0 + 2 < nc)
        def _():
            stage_idx(j0 + 2, bufA)
            fire_gathers(bufA, gsem_a)
        compute(bufB)
        fire_scatters(bufB, ssem_c)
        return ()
    lax.fori_loop(0, nc // 2, pair, ())
    drain_scatters(bufB, ssem_c)
    plsc.subcore_barrier()

    r0 = sid * nr
    pltpu.sync_copy(h_sh.at[pl.ds(r0, nr)], h_out.at[cid, pl.ds(r0, nr)])
    if with_pos:
        @pl.when(sid == 0)
        def _():
            pltpu.sync_copy(px_sh, px_o.at[pl.ds(cid * ns, ns)])
            pltpu.sync_copy(py_sh, py_o.at[pl.ds(cid * ns, ns)])
            pltpu.sync_copy(pz_sh, pz_o.at[pl.ds(cid * ns, ns)])


def _sc_conv(x, pxyz, w, d0, d1, a0, a1, src_pad, dst_pad, with_pos,
             rows_by_dst, ec=64):
    ns = x.shape[0]
    ept = src_pad.shape[0] // NW
    out_type = [jax.ShapeDtypeStruct((NCORE, ns, LD), F32)]
    out_type += [jax.ShapeDtypeStruct((NCORE * ns,), F32)] * 3
    scr = (
        [pltpu.VMEM_SHARED((ns, LD), F32)]
        + [pltpu.VMEM_SHARED((ns,), F32)] * 3
        + ([pltpu.VMEM((ec, LD), F32), pltpu.VMEM((ec,), I32),
            pltpu.VMEM((ec,), I32)] + [pltpu.VMEM((ec,), F32)] * 8) * 2
        + [pltpu.VMEM((16, LD), F32), pltpu.VMEM((ZF,), F32),
           pltpu.SemaphoreType.DMA, pltpu.SemaphoreType.DMA,
           pltpu.SemaphoreType.DMA, pltpu.SemaphoreType.DMA]
    )
    f = pl.kernel(
        functools.partial(_conv_body, ns, ept, ec, with_pos, rows_by_dst),
        out_type=out_type,
        mesh=_sc_mesh(),
        scratch_types=scr,
    )
    px, py, pz = pxyz if with_pos else (w, w, w)
    return f(x, px, py, pz, w, d0, d1, a0, a1, src_pad, dst_pad)


GCH = 80


def _compact_body(ns_out, hp0_hbm, hp1_hbm, px0, px1, py0, py1, pz0, pz1,
                  aw0, aw1, mid_hbm, h_out, px_o, py_o, pz_o, w_o,
                  mid_b, r0_b, r1_b, s0_b, s1_b):
    wid = _wid()
    nrw = ns_out // NW
    base = wid * nrw
    for j0 in range(0, nrw, GCH):
        off = base + j0
        pltpu.sync_copy(mid_hbm.at[pl.ds(off, GCH)], mid_b)
        pltpu.sync_copy(hp0_hbm.at[mid_b], r0_b)
        pltpu.sync_copy(hp1_hbm.at[mid_b], r1_b)

        def add_rows(i, _):
            for k in range(8):
                cs = pl.ds(16 * k, 16)
                r0_b[i, cs] = r0_b[i, cs] + r1_b[i, cs]
            return ()
        lax.fori_loop(0, GCH, add_rows, ())
        pltpu.sync_copy(r0_b, h_out.at[pl.ds(off, GCH)])

        for (a_, b_, o_, eps) in ((px0, px1, px_o, 0.0),
                                  (py0, py1, py_o, 0.0),
                                  (pz0, pz1, pz_o, 0.0),
                                  (aw0, aw1, w_o, 1e-12)):
            pltpu.sync_copy(a_.at[mid_b], s0_b)
            pltpu.sync_copy(b_.at[mid_b], s1_b)

            def add_s(i, _):
                cs = pl.ds(16 * i, 16)
                s0_b[cs] = s0_b[cs] + s1_b[cs] + eps
                return ()
            lax.fori_loop(0, GCH // 16, add_s, ())
            pltpu.sync_copy(s0_b, o_.at[pl.ds(off, GCH)])


def _sc_compact(hp, pxp, pyp, pzp, awp, mid_pad):
    ns_out = mid_pad.shape[0]
    f = pl.kernel(
        functools.partial(_compact_body, ns_out),
        out_type=[
            jax.ShapeDtypeStruct((ns_out, LD), F32),
            jax.ShapeDtypeStruct((ns_out,), F32),
            jax.ShapeDtypeStruct((ns_out,), F32),
            jax.ShapeDtypeStruct((ns_out,), F32),
            jax.ShapeDtypeStruct((ns_out,), F32),
        ],
        mesh=_sc_mesh(),
        scratch_types=[
            pltpu.VMEM((GCH,), I32),
            pltpu.VMEM((GCH, LD), F32),
            pltpu.VMEM((GCH, LD), F32),
            pltpu.VMEM((GCH,), F32),
            pltpu.VMEM((GCH,), F32),
        ],
    )
    return f(hp[0], hp[1], pxp[0], pxp[1], pyp[0], pyp[1], pzp[0], pzp[1],
             awp[0], awp[1], mid_pad)


def _unpool_body(ns_out, hse_hbm, sel_hbm, u_out, sel_b, r_b):
    wid = _wid()
    nrw = ns_out // NW
    base = wid * nrw
    for j0 in range(0, nrw, GCH):
        off = base + j0
        pltpu.sync_copy(sel_hbm.at[pl.ds(off, GCH)], sel_b)
        pltpu.sync_copy(hse_hbm.at[sel_b], r_b)
        pltpu.sync_copy(r_b, u_out.at[pl.ds(off, GCH)])


def _sc_unpool(h_small_ext, sel_pad):
    ns_out = sel_pad.shape[0]
    f = pl.kernel(
        functools.partial(_unpool_body, ns_out),
        out_type=jax.ShapeDtypeStruct((ns_out, LD), F32),
        mesh=_sc_mesh(),
        scratch_types=[
            pltpu.VMEM((GCH,), I32),
            pltpu.VMEM((GCH, LD), F32),
        ],
    )
    return f(h_small_ext, sel_pad)


def _pre_body(h_ref, pos_ref, wa_ref, wb_ref, wp_ref, b1_ref, u_ref, v_ref):
    h = h_ref[...]
    posp = pos_ref[...] @ wp_ref[...]
    u_ref[...] = h @ wa_ref[...] + posp + b1_ref[...]
    v_ref[...] = h @ wb_ref[...] - posp


def _tc_pre(h, posr, wa, wb, wp, b1, blk=512):
    n = h.shape[0]
    return pl.pallas_call(
        _pre_body,
        grid=(_ceil(n, blk),),
        in_specs=[
            pl.BlockSpec((blk, LD), lambda i: (i, 0)),
            pl.BlockSpec((blk, 16), lambda i: (i, 0)),
            pl.BlockSpec((LD, LD), lambda i: (0, 0)),
            pl.BlockSpec((LD, LD), lambda i: (0, 0)),
            pl.BlockSpec((16, LD), lambda i: (0, 0)),
            pl.BlockSpec((1, LD), lambda i: (0, 0)),
        ],
        out_specs=[
            pl.BlockSpec((blk, LD), lambda i: (i, 0)),
            pl.BlockSpec((blk, LD), lambda i: (i, 0)),
        ],
        out_shape=[
            jax.ShapeDtypeStruct((n, LD), F32),
            jax.ShapeDtypeStruct((n, LD), F32),
        ],
    )(h, posr, wa, wb, wp, b1)


def _node_body(h_ref, s0_ref, s1_ref, d0_ref, d1_ref, w2e_ref, b2e_ref,
               wna_ref, wnb_ref, bn1_ref, wn2_ref, bn2_ref, res_ref, o_ref):
    h = h_ref[...]
    deg = d0_ref[...] + d1_ref[...]
    aggr = (s0_ref[0] + s1_ref[0]) @ w2e_ref[...] + deg * b2e_ref[...]
    z = jnp.maximum(h @ wna_ref[...] + aggr @ wnb_ref[...] + bn1_ref[...],
                    0.0)
    o_ref[...] = h + z @ wn2_ref[...] + bn2_ref[...] + res_ref[...]


def _tc_node(h, s_part, d0, d1, w2e, b2e, wna, wnb, bn1, wn2, bn2, res,
             blk=512):
    n = h.shape[0]
    return pl.pallas_call(
        _node_body,
        grid=(_ceil(n, blk),),
        in_specs=[
            pl.BlockSpec((blk, LD), lambda i: (i, 0)),
            pl.BlockSpec((1, blk, LD), lambda i: (0, i, 0)),
            pl.BlockSpec((1, blk, LD), lambda i: (1, i, 0)),
            pl.BlockSpec((blk, 1), lambda i: (i, 0)),
            pl.BlockSpec((blk, 1), lambda i: (i, 0)),
            pl.BlockSpec((LD, LD), lambda i: (0, 0)),
            pl.BlockSpec((1, LD), lambda i: (0, 0)),
            pl.BlockSpec((LD, LD), lambda i: (0, 0)),
            pl.BlockSpec((LD, LD), lambda i: (0, 0)),
            pl.BlockSpec((1, LD), lambda i: (0, 0)),
            pl.BlockSpec((LD, LD), lambda i: (0, 0)),
            pl.BlockSpec((1, LD), lambda i: (0, 0)),
            pl.BlockSpec((blk, LD), lambda i: (i, 0)),
        ],
        out_specs=pl.BlockSpec((blk, LD), lambda i: (i, 0)),
        out_shape=jax.ShapeDtypeStruct((n, LD), F32),
    )(h, s_part, s_part, d0[:, None], d1[:, None], w2e, b2e, wna, wnb, bn1,
      wn2, bn2, res)


def _add_body(a_ref, b_ref, o_ref):
    o_ref[...] = a_ref[0] + b_ref[0]


def _tc_add2(parts, blk=512):
    n = parts.shape[1]
    return pl.pallas_call(
        _add_body,
        grid=(_ceil(n, blk),),
        in_specs=[
            pl.BlockSpec((1, blk, LD), lambda i: (0, i, 0)),
            pl.BlockSpec((1, blk, LD), lambda i: (1, i, 0)),
        ],
        out_specs=pl.BlockSpec((blk, LD), lambda i: (i, 0)),
        out_shape=jax.ShapeDtypeStruct((n, LD), F32),
    )(parts, parts)


def _split_gmp_params(p):
    w1e = p["edge"]["W1"]
    wa = w1e[:LD]
    wb = w1e[LD:2 * LD]
    wp = jnp.pad(w1e[2 * LD:2 * LD + 3], ((0, 13), (0, 0)))
    wd = w1e[2 * LD + 3]
    b1 = p["edge"]["b1"][None, :]
    w2e = p["edge"]["W2"]
    b2e = p["edge"]["b2"][None, :]
    wn1 = p["node"]["W1"]
    return (wa, wb, wp, wd, b1, w2e, b2e, wn1[:LD], wn1[LD:],
            p["node"]["b1"][None, :], p["node"]["W2"],
            p["node"]["b2"][None, :])


def _pad_edges(src, dst, n):
    e = src.shape[0]
    g = NW * ECHUNK * 2
    epad = _ceil(e, g) * g
    fill = jnp.full((epad - e,), n, I32)
    return jnp.concatenate([src, fill]), jnp.concatenate([dst, fill])


def _gmp_sc(p, h_pad, posr, pxyz, srcp, dstp, res=None):
    (wa, wb, wp, wd, b1, w2e, b2e, wna, wnb, bn1, wn2, bn2) = \
        _split_gmp_params(p)
    u, v = _tc_pre(h_pad, posr, wa, wb, wp, b1)
    ec = 64 if h_pad.shape[0] >= 10240 else 128
    s_part, dd0, dd1 = _sc_edge(u, v, pxyz, srcp, dstp, wd, ec=ec)
    if res is None:
        res = jnp.zeros_like(h_pad)
    return _tc_node(h_pad, s_part, dd0, dd1, w2e, b2e, wna, wnb, bn1, wn2,
                    bn2, res)


def _pad_rows(x, ns):
    return jnp.pad(x, ((0, ns - x.shape[0]), (0, 0)))


def _pad_flat(x, ns):
    return jnp.pad(x, (0, ns - x.shape[0]))


def kernel(h, pos, params, m_gs_0, m_gs_1, m_gs_2, m_ids_0, m_ids_1):
    m_gs = [m_gs_0, m_gs_1, m_gs_2]
    m_ids = [m_ids_0, m_ids_1]
    nlist = [10000, 5000, 2500]

    ns0 = NS_TAB[10000]
    h_pad = _pad_rows(h, ns0)
    posr = jnp.pad(pos, ((0, ns0 - pos.shape[0]), (0, 13)))
    pxyz = (posr[:, 0], posr[:, 1], posr[:, 2])
    w = _pad_flat(jnp.ones((10000,), F32), ns0)

    down_hs, down_geo, down_sc, pads = [], [], [], []
    for i in range(2):
        n = nlist[i]
        ns = NS_TAB[n]
        nsn = NS_TAB[nlist[i + 1]]
        srcp, dstp = _pad_edges(m_gs[i][0], m_gs[i][1], n)
        pads.append((srcp, dstp))
        h_pad = _gmp_sc(params["down"][i], h_pad, posr, pxyz, srcp, dstp)
        down_hs.append(h_pad)
        down_geo.append((posr, pxyz))
        d0, d1 = _sc_prep1(srcp, ns)
        a0, a1 = _sc_prep2(w, d0, d1, srcp, dstp)
        down_sc.append((w, d0, d1, a0, a1))
        ecc = 64 if ns >= 10240 else 128
        hp, pxf, pyf, pzf = _sc_conv(
            h_pad, pxyz, w, d0, d1, a0, a1, srcp, dstp, True, False, ec=ecc)
        mid_pad = _pad_flat(m_ids[i], nsn)
        h_pad, px, py, pz, w = _sc_compact(
            hp, (pxf[:ns], pxf[ns:]), (pyf[:ns], pyf[ns:]),
            (pzf[:ns], pzf[ns:]), (a0, a1), mid_pad)
        posr = jnp.concatenate(
            [px[:, None], py[:, None], pz[:, None],
             jnp.zeros((nsn, 13), F32)], axis=1)
        pxyz = (px, py, pz)

    n2 = nlist[2]
    ns2 = NS_TAB[n2]
    srcp, dstp = _pad_edges(m_gs[2][0], m_gs[2][1], n2)
    h_pad = _gmp_sc(params["bottom"], h_pad, posr, pxyz, srcp, dstp)

    for i in range(2):
        li = 1 - i
        n = nlist[li]
        ns = NS_TAB[n]
        ns_small = NS_TAB[nlist[li + 1]]
        srcp, dstp = pads[li]
        posr, pxyz = down_geo[li]
        w_l, d0, d1, a0, a1 = down_sc[li]
        hse = jnp.concatenate([h_pad, jnp.zeros((LD, LD), F32)], axis=0)
        zrow = ns_small
        sel = jnp.full((ns,), zrow, I32).at[m_ids[li]].set(
            jnp.arange(nlist[li + 1], dtype=I32))
        u_arr = _sc_unpool(hse, sel)
        ecc = 64 if ns >= 10240 else 128
        hu = _sc_conv(u_arr, None, w_l, d0, d1, a0, a1, srcp, dstp,
                      False, True, ec=ecc)[0]
        h_uc = _tc_add2(hu)
        h_pad = _gmp_sc(params["up"][i], h_uc, posr, pxyz, srcp, dstp,
                        res=down_hs[li])
    return h_pad[:10000]
